# baseline (speedup 1.0000x reference)
import functools

import jax
import jax.numpy as jnp
import numpy as np
from jax import lax
from jax.experimental import pallas as pl
from jax.experimental.pallas import tpu as pltpu

N_NODES = 10000; N_EDGES = 320000; TOKEN_DIM = 38; NUM_TOK = 30
B = 64; NC = 50; C = 5; ES = 5; EN = 10
GV = 100000; EV = 100000; WD = 300; ED = 100; D = 128; UD = 256; H = 8; HD = 16


def _encode_news(tokens, ent3, glove, Wq, Wk, Wv, W_att, b_att, v_att, W_ent):
    M = tokens.shape[0]
    e = jnp.take(glove, tokens, axis=0)
    q = (e @ Wq).reshape(M, NUM_TOK, H, HD).transpose(0, 2, 1, 3)
    k = (e @ Wk).reshape(M, NUM_TOK, H, HD).transpose(0, 2, 1, 3)
    v = (e @ Wv).reshape(M, NUM_TOK, H, HD).transpose(0, 2, 1, 3)
    att = jax.nn.softmax(jnp.einsum('mhtd,mhsd->mhts', q, k) / np.sqrt(HD), axis=-1)
    out = jnp.einsum('mhts,mhsd->mhtd', att, v).transpose(0, 2, 1, 3).reshape(M, NUM_TOK, D)
    a = jnp.tanh(out @ W_att + b_att) @ v_att
    w = jax.nn.softmax(a, axis=-1)
    pooled = jnp.einsum('mt,mtd->md', w, out)
    return pooled + jnp.mean(ent3, axis=1) @ W_ent


def _score_loss_body(cand_ref, user_ref, label_ref, loss_ref, score_ref):
    cand = cand_ref[...]
    user = user_ref[...]
    score = jnp.sum(cand * user[:, None, :], axis=-1)
    m = jnp.max(score, axis=-1, keepdims=True)
    lse = m + jnp.log(jnp.sum(jnp.exp(score - m), axis=-1, keepdims=True))
    logp = score - lse
    lbl = label_ref[...]
    onehot = (lax.broadcasted_iota(jnp.int32, (B, C), 1) == lbl[:, None]).astype(jnp.float32)
    loss = -jnp.mean(jnp.sum(logp * onehot, axis=-1))
    loss_ref[...] = jnp.broadcast_to(loss, (1, 1))
    score_ref[...] = score


def _score_loss(cand_final, user_emb, label):
    loss, score = pl.pallas_call(
        _score_loss_body,
        out_shape=(jax.ShapeDtypeStruct((1, 1), jnp.float32),
                   jax.ShapeDtypeStruct((B, C), jnp.float32)),
    )(cand_final, user_emb, label)
    return loss[0, 0], score


def kernel(subgraph_x, edge_index, mapping_idx, candidate_news, candidate_entity, entity_mask, label,
           glove, entity_table, Wq, Wk, Wv, W_att, b_att, v_att, W_ent, W_ggc,
           W_ih, W_hh, b_ih, b_hh, W_lih, W_lhh, b_l, W_ge, b_ge, v_ge, W_gproj):
    all_entity = subgraph_x[:, -8:-3]
    ent = jnp.take(entity_table, all_entity, axis=0)
    ent3 = jnp.concatenate([ent, ent, ent], axis=-1)
    x_encoded = _encode_news(subgraph_x[:, :NUM_TOK], ent3, glove, Wq, Wk, Wv, W_att, b_att, v_att, W_ent)
    src, dst = edge_index[0], edge_index[1]
    h = x_encoded
    for l in range(3):
        m = h @ W_ggc[l]
        agg = jax.ops.segment_sum(jnp.take(m, src, axis=0), dst, num_segments=N_NODES)
        gi = agg @ W_ih + b_ih
        gh = h @ W_hh + b_hh
        r = jax.nn.sigmoid(gi[:, :D] + gh[:, :D])
        z = jax.nn.sigmoid(gi[:, D:2 * D] + gh[:, D:2 * D])
        n = jnp.tanh(gi[:, 2 * D:] + r * gh[:, 2 * D:])
        h = (1.0 - z) * n + z * h
    graph_emb = h
    clicked = jnp.concatenate([jnp.take(x_encoded, mapping_idx, axis=0),
                               jnp.take(graph_emb, mapping_idx, axis=0)], axis=-1)

    def lstm_step(carry, xt):
        hs, cs = carry
        g = xt @ W_lih + hs @ W_lhh + b_l
        i = jax.nn.sigmoid(g[:, :UD]); f = jax.nn.sigmoid(g[:, UD:2 * UD])
        gg = jnp.tanh(g[:, 2 * UD:3 * UD]); o = jax.nn.sigmoid(g[:, 3 * UD:])
        cs = f * cs + i * gg
        hs = o * jnp.tanh(cs)
        return (hs, cs), None
    (user_emb, _), _ = lax.scan(lstm_step,
                                (jnp.zeros((B, UD), jnp.float32), jnp.zeros((B, UD), jnp.float32)),
                                clicked.transpose(1, 0, 2))
    origin_e = candidate_entity[..., :ES]
    neighbor_e = candidate_entity[..., ES:]
    oe = jnp.take(entity_table, origin_e, axis=0)
    oe3 = jnp.concatenate([oe, oe, oe], axis=-1).reshape(B * C, ES, 3 * ED)
    cand_title = _encode_news(candidate_news[..., :NUM_TOK].reshape(B * C, NUM_TOK), oe3,
                              glove, Wq, Wk, Wv, W_att, b_att, v_att, W_ent).reshape(B, C, D)
    ne = jnp.take(entity_table, neighbor_e, axis=0)
    a = jnp.tanh(ne @ W_ge + b_ge) @ v_ge
    a = jnp.where(entity_mask > 0, a, -1e9)
    w = jax.nn.softmax(a, axis=-1)
    cand_nb = jnp.einsum('bcn,bcnd->bcd', w, ne) @ W_gproj
    cand_final = jnp.concatenate([cand_nb, cand_title], axis=-1)
    loss, score = _score_loss(cand_final, user_emb, label)
    return (loss, score)


# speedup vs baseline: 1.2680x; 1.2680x over previous
import functools

import jax
import jax.numpy as jnp
import numpy as np
from jax import lax
from jax.experimental import pallas as pl
from jax.experimental.pallas import tpu as pltpu

N_NODES = 10000; N_EDGES = 320000; TOKEN_DIM = 38; NUM_TOK = 30
B = 64; NC = 50; C = 5; ES = 5; EN = 10
GV = 100000; EV = 100000; WD = 300; ED = 100; D = 128; UD = 256; H = 8; HD = 16
TPAD = 32
WPAD = 304
EPAD = 112
NB = 16


def _encode_body(e_ref, entm_ref, wqkv_ref, watt_ref, batt_ref, vatt_ref,
                 went_ref, wggc0_ref, xenc_ref, m0_ref):
    e = e_ref[...]
    rowmod = lax.broadcasted_iota(jnp.int32, (NB * TPAD, WPAD), 0) % TPAD
    e = jnp.where(rowmod < NUM_TOK, e, 0.0)
    qkv = jnp.dot(e, wqkv_ref[...], preferred_element_type=jnp.float32)

    i0 = lax.broadcasted_iota(jnp.int32, (H * TPAD, D), 0)
    i1 = lax.broadcasted_iota(jnp.int32, (H * TPAD, D), 1)
    headmask = (i0 // TPAD == i1 // HD).astype(jnp.float32)
    o0 = lax.broadcasted_iota(jnp.int32, (H * TPAD, H), 0)
    o1 = lax.broadcasted_iota(jnp.int32, (H * TPAD, H), 1)
    onesbd = ((o0 // TPAD == o1) & (o0 % TPAD < NUM_TOK)).astype(jnp.float32)
    x0 = lax.broadcasted_iota(jnp.int32, (H, D), 0)
    x1 = lax.broadcasted_iota(jnp.int32, (H, D), 1)
    expand = (x0 == x1 // HD).astype(jnp.float32)
    tmask = lax.broadcasted_iota(jnp.int32, (TPAD, 1), 0) < NUM_TOK

    watt = watt_ref[...]; batt = batt_ref[...]; vatt = vatt_ref[...]
    pooled_rows = []
    for n in range(NB):
        q = qkv[n * TPAD:(n + 1) * TPAD, 0:D]
        k = qkv[n * TPAD:(n + 1) * TPAD, D:2 * D]
        v = qkv[n * TPAD:(n + 1) * TPAD, 2 * D:3 * D]
        khat = jnp.concatenate([k] * H, axis=0) * headmask
        vhat = jnp.concatenate([v] * H, axis=0) * headmask
        scores = lax.dot_general(q, khat, (((1,), (1,)), ((), ())),
                                 preferred_element_type=jnp.float32) * 0.25
        mrow = jnp.max(scores, axis=1, keepdims=True)
        es = jnp.exp(scores - mrow)
        denom = lax.dot_general(es, onesbd, (((1,), (0,)), ((), ())),
                                preferred_element_type=jnp.float32)
        recipb = lax.dot_general(1.0 / denom, expand, (((1,), (0,)), ((), ())),
                                 preferred_element_type=jnp.float32)
        outn = lax.dot_general(es, vhat, (((1,), (0,)), ((), ())),
                               preferred_element_type=jnp.float32) * recipb
        an = jnp.tanh(jnp.dot(outn, watt, preferred_element_type=jnp.float32) + batt)
        al = jnp.sum(an * vatt, axis=1, keepdims=True)
        al = jnp.where(tmask, al, -1e30)
        wm = jnp.exp(al - jnp.max(al, axis=0, keepdims=True))
        wm = wm / jnp.sum(wm, axis=0, keepdims=True)
        pooled_rows.append(jnp.sum(outn * wm, axis=0, keepdims=True))
    pooledb = jnp.concatenate(pooled_rows, axis=0)
    entm = jnp.mean(entm_ref[...], axis=1)
    went = went_ref[0] + went_ref[1] + went_ref[2]
    xe = pooledb + jnp.dot(entm, went, preferred_element_type=jnp.float32)
    xenc_ref[...] = xe
    m0_ref[...] = jnp.dot(xe, wggc0_ref[...], preferred_element_type=jnp.float32)


def _encode_pallas(e_flat, entm, Wqkv, W_att, b_att, v_att, W_ent3, W_ggc0):
    M = entm.shape[0]
    grid = M // NB
    return pl.pallas_call(
        _encode_body,
        grid=(grid,),
        in_specs=[
            pl.BlockSpec((NB * TPAD, WPAD), lambda i: (i, 0)),
            pl.BlockSpec((NB, 5, EPAD), lambda i: (i, 0, 0)),
            pl.BlockSpec((WPAD, 3 * D), lambda i: (0, 0)),
            pl.BlockSpec((D, D), lambda i: (0, 0)),
            pl.BlockSpec((1, D), lambda i: (0, 0)),
            pl.BlockSpec((1, D), lambda i: (0, 0)),
            pl.BlockSpec((3, EPAD, D), lambda i: (0, 0, 0)),
            pl.BlockSpec((D, D), lambda i: (0, 0)),
        ],
        out_specs=[
            pl.BlockSpec((NB, D), lambda i: (i, 0)),
            pl.BlockSpec((NB, D), lambda i: (i, 0)),
        ],
        out_shape=[
            jax.ShapeDtypeStruct((M, D), jnp.float32),
            jax.ShapeDtypeStruct((M, D), jnp.float32),
        ],
    )(e_flat, entm, Wqkv, W_att, b_att, v_att, W_ent3, W_ggc0)


GRU_BLK = 400


def _gru_body(h_ref, agg_ref, wih_ref, whh_ref, bih_ref, bhh_ref, wnext_ref,
              hout_ref, mnext_ref):
    h = h_ref[...]
    agg = agg_ref[...]
    gi = jnp.dot(agg, wih_ref[...], preferred_element_type=jnp.float32) + bih_ref[...]
    gh = jnp.dot(h, whh_ref[...], preferred_element_type=jnp.float32) + bhh_ref[...]
    r = jax.nn.sigmoid(gi[:, :D] + gh[:, :D])
    z = jax.nn.sigmoid(gi[:, D:2 * D] + gh[:, D:2 * D])
    n = jnp.tanh(gi[:, 2 * D:] + r * gh[:, 2 * D:])
    hn = (1.0 - z) * n + z * h
    hout_ref[...] = hn
    mnext_ref[...] = jnp.dot(hn, wnext_ref[...], preferred_element_type=jnp.float32)


def _gru_pallas(h, agg, W_ih, W_hh, b_ih, b_hh, W_next):
    M = h.shape[0]
    return pl.pallas_call(
        _gru_body,
        grid=(M // GRU_BLK,),
        in_specs=[
            pl.BlockSpec((GRU_BLK, D), lambda i: (i, 0)),
            pl.BlockSpec((GRU_BLK, D), lambda i: (i, 0)),
            pl.BlockSpec((D, 3 * D), lambda i: (0, 0)),
            pl.BlockSpec((D, 3 * D), lambda i: (0, 0)),
            pl.BlockSpec((1, 3 * D), lambda i: (0, 0)),
            pl.BlockSpec((1, 3 * D), lambda i: (0, 0)),
            pl.BlockSpec((D, D), lambda i: (0, 0)),
        ],
        out_specs=[
            pl.BlockSpec((GRU_BLK, D), lambda i: (i, 0)),
            pl.BlockSpec((GRU_BLK, D), lambda i: (i, 0)),
        ],
        out_shape=[
            jax.ShapeDtypeStruct((M, D), jnp.float32),
            jax.ShapeDtypeStruct((M, D), jnp.float32),
        ],
    )(h, agg, W_ih, W_hh, b_ih, b_hh, W_next)


def _lstm_body(x_ref, wih_ref, whh_ref, b_ref, hout_ref):
    wih = wih_ref[...]; whh = whh_ref[...]; bb = b_ref[...]

    def step(t, carry):
        hs, cs = carry
        xt = x_ref[pl.ds(t * B, B), :]
        g = (jnp.dot(xt, wih, preferred_element_type=jnp.float32)
             + jnp.dot(hs, whh, preferred_element_type=jnp.float32) + bb)
        i = jax.nn.sigmoid(g[:, :UD])
        f = jax.nn.sigmoid(g[:, UD:2 * UD])
        gg = jnp.tanh(g[:, 2 * UD:3 * UD])
        o = jax.nn.sigmoid(g[:, 3 * UD:])
        cs = f * cs + i * gg
        hs = o * jnp.tanh(cs)
        return (hs, cs)

    z = jnp.zeros((B, UD), jnp.float32)
    hs, _ = lax.fori_loop(0, NC, step, (z, z))
    hout_ref[...] = hs


def _lstm_pallas(clicked_tm, W_lih, W_lhh, b_l):
    return pl.pallas_call(
        _lstm_body,
        out_shape=jax.ShapeDtypeStruct((B, UD), jnp.float32),
    )(clicked_tm, W_lih, W_lhh, b_l)


NBC = 8
NE = ES * EN


def _candnb_body(ne_ref, maskt_ref, wge_ref, bge_ref, vge_ref, wproj_ref, out_ref):
    ne = ne_ref[...]
    t = jnp.tanh(jnp.dot(ne, wge_ref[...], preferred_element_type=jnp.float32)
                 + bge_ref[...])
    a = jnp.sum(t * vge_ref[...], axis=1, keepdims=True)
    rows = []
    for c in range(NBC):
        ac = a[c * NE:(c + 1) * NE, :]
        mc = maskt_ref[0, :, c:c + 1]
        ac = jnp.where(mc > 0, ac, -1e9)
        wme = jnp.exp(ac - jnp.max(ac, axis=0, keepdims=True))
        wme = wme / jnp.sum(wme, axis=0, keepdims=True)
        nec = ne[c * NE:(c + 1) * NE, :]
        rows.append(jnp.sum(nec * wme, axis=0, keepdims=True))
    wsum = jnp.concatenate(rows, axis=0)
    out_ref[...] = jnp.dot(wsum, wproj_ref[...], preferred_element_type=jnp.float32)


def _candnb_pallas(ne_rows, mask_t, W_ge, b_ge, v_ge, W_gproj):
    M = B * C
    return pl.pallas_call(
        _candnb_body,
        grid=(M // NBC,),
        in_specs=[
            pl.BlockSpec((NBC * NE, EPAD), lambda i: (i, 0)),
            pl.BlockSpec((1, NE, NBC), lambda i: (i, 0, 0)),
            pl.BlockSpec((EPAD, EPAD), lambda i: (0, 0)),
            pl.BlockSpec((1, EPAD), lambda i: (0, 0)),
            pl.BlockSpec((1, EPAD), lambda i: (0, 0)),
            pl.BlockSpec((EPAD, D), lambda i: (0, 0)),
        ],
        out_specs=pl.BlockSpec((NBC, D), lambda i: (i, 0)),
        out_shape=jax.ShapeDtypeStruct((M, D), jnp.float32),
    )(ne_rows, mask_t, W_ge, b_ge, v_ge, W_gproj)


def _score_loss_body(cand_ref, user_ref, label_ref, loss_ref, score_ref):
    cand = cand_ref[...]
    user = user_ref[...]
    score = jnp.sum(cand * user[:, None, :], axis=-1)
    m = jnp.max(score, axis=-1, keepdims=True)
    lse = m + jnp.log(jnp.sum(jnp.exp(score - m), axis=-1, keepdims=True))
    logp = score - lse
    lbl = label_ref[...]
    onehot = (lax.broadcasted_iota(jnp.int32, (B, C), 1) == lbl[:, None]).astype(jnp.float32)
    loss = -jnp.mean(jnp.sum(logp * onehot, axis=-1))
    loss_ref[...] = jnp.broadcast_to(loss, (1, 1))
    score_ref[...] = score


def _score_loss(cand_final, user_emb, label):
    loss, score = pl.pallas_call(
        _score_loss_body,
        out_shape=(jax.ShapeDtypeStruct((1, 1), jnp.float32),
                   jax.ShapeDtypeStruct((B, C), jnp.float32)),
    )(cand_final, user_emb, label)
    return loss[0, 0], score


def kernel(subgraph_x, edge_index, mapping_idx, candidate_news, candidate_entity, entity_mask, label,
           glove, entity_table, Wq, Wk, Wv, W_att, b_att, v_att, W_ent, W_ggc,
           W_ih, W_hh, b_ih, b_hh, W_lih, W_lhh, b_l, W_ge, b_ge, v_ge, W_gproj):
    f32 = jnp.float32
    glove_pad = jnp.pad(glove, ((0, 0), (0, WPAD - WD)))
    ent_pad = jnp.pad(entity_table, ((0, 0), (0, EPAD - ED)))
    tokens_all = jnp.concatenate(
        [subgraph_x[:, :NUM_TOK],
         candidate_news[..., :NUM_TOK].reshape(B * C, NUM_TOK)], axis=0)
    tok_pad = jnp.pad(tokens_all, ((0, 0), (0, TPAD - NUM_TOK)))
    ent_ids = jnp.concatenate(
        [subgraph_x[:, -8:-3],
         candidate_entity[..., :ES].reshape(B * C, ES)], axis=0)
    Wqkv = jnp.pad(jnp.concatenate([Wq, Wk, Wv], axis=1), ((0, WPAD - WD), (0, 0)))
    W_ent3 = jnp.pad(W_ent.reshape(3, ED, D), ((0, 0), (0, EPAD - ED), (0, 0)))

    e_flat = jnp.take(glove_pad, tok_pad.reshape(-1), axis=0)
    entm = jnp.take(ent_pad, ent_ids, axis=0)

    x_all, m0_all = _encode_pallas(e_flat, entm, Wqkv, W_att,
                                   b_att.reshape(1, D), v_att.reshape(1, D),
                                   W_ent3, W_ggc[0])
    x_encoded = x_all[:N_NODES]
    cand_title = x_all[N_NODES:].reshape(B, C, D)

    src, dst = edge_index[0], edge_index[1]
    h = x_encoded
    m = m0_all[:N_NODES]
    for l in range(3):
        agg = jax.ops.segment_sum(jnp.take(m, src, axis=0), dst, num_segments=N_NODES)
        W_next = W_ggc[l + 1] if l < 2 else jnp.zeros((D, D), f32)
        h, m = _gru_pallas(h, agg, W_ih, W_hh, b_ih.reshape(1, 3 * D),
                           b_hh.reshape(1, 3 * D), W_next)
    graph_emb = h

    flat_map = mapping_idx.T.reshape(-1)
    clicked_tm = jnp.concatenate([jnp.take(x_encoded, flat_map, axis=0),
                                  jnp.take(graph_emb, flat_map, axis=0)], axis=-1)
    user_emb = _lstm_pallas(clicked_tm, W_lih, W_lhh, b_l.reshape(1, 4 * UD))

    neighbor_e = candidate_entity[..., ES:].reshape(B * C * NE)
    ne_rows = jnp.take(ent_pad, neighbor_e, axis=0)
    mask_t = entity_mask.reshape(B * C // NBC, NBC, NE).transpose(0, 2, 1)
    W_ge_pad = jnp.pad(W_ge, ((0, EPAD - ED), (0, EPAD - ED)))
    b_ge_pad = jnp.pad(b_ge, (0, EPAD - ED)).reshape(1, EPAD)
    v_ge_pad = jnp.pad(v_ge, (0, EPAD - ED)).reshape(1, EPAD)
    W_gproj_pad = jnp.pad(W_gproj, ((0, EPAD - ED), (0, 0)))
    cand_nb = _candnb_pallas(ne_rows, mask_t, W_ge_pad, b_ge_pad, v_ge_pad,
                             W_gproj_pad).reshape(B, C, D)

    cand_final = jnp.concatenate([cand_nb, cand_title], axis=-1)
    loss, score = _score_loss(cand_final, user_emb, label)
    return (loss, score)


# speedup vs baseline: 2.0530x; 1.6191x over previous
import functools

import jax
import jax.numpy as jnp
import numpy as np
from jax import lax
from jax.experimental import pallas as pl
from jax.experimental.pallas import tpu as pltpu
from jax.experimental.pallas import tpu_sc as plsc

N_NODES = 10000; N_EDGES = 320000; TOKEN_DIM = 38; NUM_TOK = 30
B = 64; NC = 50; C = 5; ES = 5; EN = 10
GV = 100000; EV = 100000; WD = 300; ED = 100; D = 128; UD = 256; H = 8; HD = 16
TPAD = 32
EPAD = 128
QKVW = 3 * D
NB = 16
NE = ES * EN

NSC = 2; NTILE = 16; NW = NSC * NTILE
M_ALL = N_NODES + B * C
TOKROWS = M_ALL * TPAD
TOK_NCH, TOK_CH = 86, 120
ENTROWS = 53760
ENT_NCH, ENT_CH = 14, 120
NBROWS = B * C * NE
NBROWS_P = 16384
NBE_NCH, NBE_CH = 4, 128
E_NCH, E_CH = 125, 80
ACC_ROWS = 10240
NODES_PER_TILE = ACC_ROWS // NTILE
PREP_BLK = 400


def _prep_body(g_ref, e_ref, wqkv_ref, went_ref, qkv_ref, entp_ref):
    went = went_ref[0:ED] + went_ref[ED:2 * ED] + went_ref[2 * ED:3 * ED]
    qkv_ref[...] = jnp.dot(g_ref[...], wqkv_ref[...],
                           preferred_element_type=jnp.float32)
    entp_ref[...] = jnp.dot(e_ref[...], went,
                            preferred_element_type=jnp.float32)


def _prep_pallas(glove, entity_table, Wqkv, W_ent):
    return pl.pallas_call(
        _prep_body,
        grid=(GV // PREP_BLK,),
        in_specs=[
            pl.BlockSpec((PREP_BLK, WD), lambda i: (i, 0)),
            pl.BlockSpec((PREP_BLK, ED), lambda i: (i, 0)),
            pl.BlockSpec((WD, QKVW), lambda i: (0, 0)),
            pl.BlockSpec((3 * ED, D), lambda i: (0, 0)),
        ],
        out_specs=[
            pl.BlockSpec((PREP_BLK, QKVW), lambda i: (i, 0)),
            pl.BlockSpec((PREP_BLK, D), lambda i: (i, 0)),
        ],
        out_shape=[
            jax.ShapeDtypeStruct((GV, QKVW), jnp.float32),
            jax.ShapeDtypeStruct((EV, D), jnp.float32),
        ],
    )(glove, entity_table, Wqkv, W_ent)


def _sc_gather_all(glove_qkv, ent_proj, ent_pad, tok_idx3, ent_idx3, nb_idx3):
    mesh = plsc.VectorSubcoreMesh(core_axis_name="c", subcore_axis_name="s")

    @functools.partial(
        pl.kernel,
        out_type=[jax.ShapeDtypeStruct((TOKROWS, QKVW), jnp.float32),
                  jax.ShapeDtypeStruct((ENTROWS, D), jnp.float32),
                  jax.ShapeDtypeStruct((NBROWS_P, EPAD), jnp.float32)],
        mesh=mesh,
        scratch_types=[pltpu.VMEM((TOK_NCH, TOK_CH), jnp.int32),
                       pltpu.VMEM((ENT_NCH, ENT_CH), jnp.int32),
                       pltpu.VMEM((NBE_NCH, NBE_CH), jnp.int32),
                       pltpu.VMEM((TOK_CH, QKVW), jnp.float32),
                       pltpu.VMEM((ENT_CH, D), jnp.float32),
                       pltpu.VMEM((NBE_CH, EPAD), jnp.float32),
                       pltpu.SemaphoreType.DMA],
    )
    def k(gq_hbm, ep_hbm, er_hbm, tokidx_hbm, entidx_hbm, nbidx_hbm,
          qkvout_hbm, entout_hbm, nbout_hbm,
          tokidx_v, entidx_v, nbidx_v, tokbuf, entbuf, nbbuf, sem):
        cid = lax.axis_index("c"); sid = lax.axis_index("s")
        wid = sid * NSC + cid
        pltpu.sync_copy(tokidx_hbm.at[wid], tokidx_v)
        pltpu.sync_copy(entidx_hbm.at[wid], entidx_v)
        pltpu.sync_copy(nbidx_hbm.at[wid], nbidx_v)
        tbase = wid * (TOK_NCH * TOK_CH)

        @pl.loop(0, TOK_NCH)
        def _tok(j):
            pltpu.async_copy(gq_hbm.at[tokidx_v.at[j]], tokbuf, sem).wait()
            pltpu.sync_copy(tokbuf, qkvout_hbm.at[pl.ds(tbase + j * TOK_CH, TOK_CH)])

        ebase = wid * (ENT_NCH * ENT_CH)

        @pl.loop(0, ENT_NCH)
        def _ent(j):
            pltpu.async_copy(ep_hbm.at[entidx_v.at[j]], entbuf, sem).wait()
            pltpu.sync_copy(entbuf, entout_hbm.at[pl.ds(ebase + j * ENT_CH, ENT_CH)])

        nbase = wid * (NBE_NCH * NBE_CH)

        @pl.loop(0, NBE_NCH)
        def _nb(j):
            pltpu.async_copy(er_hbm.at[nbidx_v.at[j]], nbbuf, sem).wait()
            pltpu.sync_copy(nbbuf, nbout_hbm.at[pl.ds(nbase + j * NBE_CH, NBE_CH)])

    return k(glove_qkv, ent_proj, ent_pad, tok_idx3, ent_idx3, nb_idx3)


def _sc_segsum(m, srcidx3, dstidx3, zrows):
    mesh = plsc.VectorSubcoreMesh(core_axis_name="c", subcore_axis_name="s")

    @functools.partial(
        pl.kernel,
        out_type=jax.ShapeDtypeStruct((NSC, ACC_ROWS, D), jnp.float32),
        mesh=mesh,
        scratch_types=[pltpu.VMEM((E_NCH, E_CH), jnp.int32),
                       pltpu.VMEM((E_NCH, E_CH), jnp.int32),
                       pltpu.VMEM((E_CH, D), jnp.float32),
                       pltpu.VMEM_SHARED((ACC_ROWS, D), jnp.float32),
                       pltpu.SemaphoreType.DMA],
    )
    def k(m_hbm, srcidx_hbm, dstidx_hbm, z_hbm, out_hbm,
          sidx_v, didx_v, rows_v, acc, sem):
        cid = lax.axis_index("c"); sid = lax.axis_index("s")
        wid2 = cid * NTILE + sid
        pltpu.sync_copy(srcidx_hbm.at[wid2], sidx_v)
        pltpu.sync_copy(dstidx_hbm.at[wid2], didx_v)
        pltpu.sync_copy(z_hbm, acc.at[pl.ds(sid * NODES_PER_TILE, NODES_PER_TILE)])
        plsc.subcore_barrier()

        @pl.loop(0, E_NCH)
        def _e(j):
            pltpu.async_copy(m_hbm.at[sidx_v.at[j]], rows_v, sem).wait()
            pltpu.sync_copy(rows_v, acc.at[didx_v.at[j]], add=True)

        plsc.subcore_barrier()
        pltpu.sync_copy(
            acc.at[pl.ds(sid * NODES_PER_TILE, NODES_PER_TILE)],
            out_hbm.at[cid].at[pl.ds(sid * NODES_PER_TILE, NODES_PER_TILE)])

    return k(m, srcidx3, dstidx3, zrows)


CLK = NC * B
CLK_PT = 128
CLK_TILES = CLK // CLK_PT


def _sc_gather_clicked(xenc, gemb, map_idx3):
    mesh = plsc.VectorSubcoreMesh(core_axis_name="c", subcore_axis_name="s")

    @functools.partial(
        pl.kernel,
        out_type=[jax.ShapeDtypeStruct((CLK, D), jnp.float32),
                  jax.ShapeDtypeStruct((CLK, D), jnp.float32)],
        mesh=mesh,
        scratch_types=[pltpu.VMEM((1, CLK_PT), jnp.int32),
                       pltpu.VMEM((CLK_PT, D), jnp.float32),
                       pltpu.SemaphoreType.DMA],
    )
    def k(xenc_hbm, gemb_hbm, mapidx_hbm, out1_hbm, out2_hbm, idx_v, buf, sem):
        cid = lax.axis_index("c"); sid = lax.axis_index("s")
        wid = sid * NSC + cid

        @pl.when(wid < CLK_TILES)
        def _():
            pltpu.sync_copy(mapidx_hbm.at[wid], idx_v)
            pltpu.async_copy(xenc_hbm.at[idx_v.at[0]], buf, sem).wait()
            pltpu.sync_copy(buf, out1_hbm.at[pl.ds(wid * CLK_PT, CLK_PT)])
            pltpu.async_copy(gemb_hbm.at[idx_v.at[0]], buf, sem).wait()
            pltpu.sync_copy(buf, out2_hbm.at[pl.ds(wid * CLK_PT, CLK_PT)])

    return k(xenc, gemb, map_idx3)


def _encode_body(qkv_ref, entm_ref, watt_ref, batt_ref, vatt_ref,
                 wggc0_ref, xenc_ref, m0_ref):
    qkv = qkv_ref[...]
    rowmod = lax.broadcasted_iota(jnp.int32, (NB * TPAD, QKVW), 0) % TPAD
    qkv = jnp.where(rowmod < NUM_TOK, qkv, 0.0)

    i0 = lax.broadcasted_iota(jnp.int32, (H * TPAD, D), 0)
    i1 = lax.broadcasted_iota(jnp.int32, (H * TPAD, D), 1)
    headmask = (i0 // TPAD == i1 // HD).astype(jnp.float32)
    o0 = lax.broadcasted_iota(jnp.int32, (H * TPAD, H), 0)
    o1 = lax.broadcasted_iota(jnp.int32, (H * TPAD, H), 1)
    onesbd = ((o0 // TPAD == o1) & (o0 % TPAD < NUM_TOK)).astype(jnp.float32)
    x0 = lax.broadcasted_iota(jnp.int32, (H, D), 0)
    x1 = lax.broadcasted_iota(jnp.int32, (H, D), 1)
    expand = (x0 == x1 // HD).astype(jnp.float32)
    tmask = lax.broadcasted_iota(jnp.int32, (TPAD, 1), 0) < NUM_TOK

    watt = watt_ref[...]; batt = batt_ref[...]; vatt = vatt_ref[...]
    pooled_rows = []
    for n in range(NB):
        q = qkv[n * TPAD:(n + 1) * TPAD, 0:D]
        k = qkv[n * TPAD:(n + 1) * TPAD, D:2 * D]
        v = qkv[n * TPAD:(n + 1) * TPAD, 2 * D:3 * D]
        khat = jnp.concatenate([k] * H, axis=0) * headmask
        vhat = jnp.concatenate([v] * H, axis=0) * headmask
        scores = lax.dot_general(q, khat, (((1,), (1,)), ((), ())),
                                 preferred_element_type=jnp.float32) * 0.25
        mrow = jnp.max(scores, axis=1, keepdims=True)
        es = jnp.exp(scores - mrow)
        denom = lax.dot_general(es, onesbd, (((1,), (0,)), ((), ())),
                                preferred_element_type=jnp.float32)
        recipb = lax.dot_general(1.0 / denom, expand, (((1,), (0,)), ((), ())),
                                 preferred_element_type=jnp.float32)
        outn = lax.dot_general(es, vhat, (((1,), (0,)), ((), ())),
                               preferred_element_type=jnp.float32) * recipb
        an = jnp.tanh(jnp.dot(outn, watt, preferred_element_type=jnp.float32) + batt)
        al = jnp.sum(an * vatt, axis=1, keepdims=True)
        al = jnp.where(tmask, al, -1e30)
        wm = jnp.exp(al - jnp.max(al, axis=0, keepdims=True))
        wm = wm / jnp.sum(wm, axis=0, keepdims=True)
        pooled_rows.append(jnp.sum(outn * wm, axis=0, keepdims=True))
    pooledb = jnp.concatenate(pooled_rows, axis=0)
    entm = jnp.mean(entm_ref[...], axis=1)
    xe = pooledb + entm
    xenc_ref[...] = xe
    m0_ref[...] = jnp.dot(xe, wggc0_ref[...], preferred_element_type=jnp.float32)


def _encode_pallas(qkv_flat, entm, W_att, b_att, v_att, W_ggc0):
    M = entm.shape[0]
    grid = M // NB
    return pl.pallas_call(
        _encode_body,
        grid=(grid,),
        in_specs=[
            pl.BlockSpec((NB * TPAD, QKVW), lambda i: (i, 0)),
            pl.BlockSpec((NB, ES, D), lambda i: (i, 0, 0)),
            pl.BlockSpec((D, D), lambda i: (0, 0)),
            pl.BlockSpec((1, D), lambda i: (0, 0)),
            pl.BlockSpec((1, D), lambda i: (0, 0)),
            pl.BlockSpec((D, D), lambda i: (0, 0)),
        ],
        out_specs=[
            pl.BlockSpec((NB, D), lambda i: (i, 0)),
            pl.BlockSpec((NB, D), lambda i: (i, 0)),
        ],
        out_shape=[
            jax.ShapeDtypeStruct((M, D), jnp.float32),
            jax.ShapeDtypeStruct((M, D), jnp.float32),
        ],
    )(qkv_flat, entm, W_att, b_att, v_att, W_ggc0)


GRU_BLK = 400


def _gru_body(h_ref, agg0_ref, agg1_ref, wih_ref, whh_ref, bih_ref, bhh_ref, wnext_ref,
              hout_ref, mnext_ref):
    h = h_ref[...]
    agg = agg0_ref[0] + agg1_ref[0]
    gi = jnp.dot(agg, wih_ref[...], preferred_element_type=jnp.float32) + bih_ref[...]
    gh = jnp.dot(h, whh_ref[...], preferred_element_type=jnp.float32) + bhh_ref[...]
    r = jax.nn.sigmoid(gi[:, :D] + gh[:, :D])
    z = jax.nn.sigmoid(gi[:, D:2 * D] + gh[:, D:2 * D])
    n = jnp.tanh(gi[:, 2 * D:] + r * gh[:, 2 * D:])
    hn = (1.0 - z) * n + z * h
    hout_ref[...] = hn
    mnext_ref[...] = jnp.dot(hn, wnext_ref[...], preferred_element_type=jnp.float32)


def _gru_pallas(h, agg2, W_ih, W_hh, b_ih, b_hh, W_next):
    M = h.shape[0]
    return pl.pallas_call(
        _gru_body,
        grid=(M // GRU_BLK,),
        in_specs=[
            pl.BlockSpec((GRU_BLK, D), lambda i: (i, 0)),
            pl.BlockSpec((1, GRU_BLK, D), lambda i: (0, i, 0)),
            pl.BlockSpec((1, GRU_BLK, D), lambda i: (1, i, 0)),
            pl.BlockSpec((D, 3 * D), lambda i: (0, 0)),
            pl.BlockSpec((D, 3 * D), lambda i: (0, 0)),
            pl.BlockSpec((1, 3 * D), lambda i: (0, 0)),
            pl.BlockSpec((1, 3 * D), lambda i: (0, 0)),
            pl.BlockSpec((D, D), lambda i: (0, 0)),
        ],
        out_specs=[
            pl.BlockSpec((GRU_BLK, D), lambda i: (i, 0)),
            pl.BlockSpec((GRU_BLK, D), lambda i: (i, 0)),
        ],
        out_shape=[
            jax.ShapeDtypeStruct((M, D), jnp.float32),
            jax.ShapeDtypeStruct((M, D), jnp.float32),
        ],
    )(h, agg2, agg2, W_ih, W_hh, b_ih, b_hh, W_next)


def _lstm_body(x_ref, wih_ref, whh_ref, b_ref, hout_ref):
    wih = wih_ref[...]; whh = whh_ref[...]; bb = b_ref[...]

    def step(t, carry):
        hs, cs = carry
        xt = x_ref[pl.ds(t * B, B), :]
        g = (jnp.dot(xt, wih, preferred_element_type=jnp.float32)
             + jnp.dot(hs, whh, preferred_element_type=jnp.float32) + bb)
        i = jax.nn.sigmoid(g[:, :UD])
        f = jax.nn.sigmoid(g[:, UD:2 * UD])
        gg = jnp.tanh(g[:, 2 * UD:3 * UD])
        o = jax.nn.sigmoid(g[:, 3 * UD:])
        cs = f * cs + i * gg
        hs = o * jnp.tanh(cs)
        return (hs, cs)

    z = jnp.zeros((B, UD), jnp.float32)
    hs, _ = lax.fori_loop(0, NC, step, (z, z))
    hout_ref[...] = hs


def _lstm_pallas(clicked_tm, W_lih, W_lhh, b_l):
    return pl.pallas_call(
        _lstm_body,
        out_shape=jax.ShapeDtypeStruct((B, UD), jnp.float32),
    )(clicked_tm, W_lih, W_lhh, b_l)


NBC = 8


def _candnb_body(ne_ref, maskt_ref, wge_ref, bge_ref, vge_ref, wproj_ref, out_ref):
    ne = ne_ref[...]
    t = jnp.tanh(jnp.dot(ne, wge_ref[...], preferred_element_type=jnp.float32)
                 + bge_ref[...])
    a = jnp.sum(t * vge_ref[...], axis=1, keepdims=True)
    rows = []
    for c in range(NBC):
        ac = a[c * NE:(c + 1) * NE, :]
        mc = maskt_ref[0, :, c:c + 1]
        ac = jnp.where(mc > 0, ac, -1e9)
        wme = jnp.exp(ac - jnp.max(ac, axis=0, keepdims=True))
        wme = wme / jnp.sum(wme, axis=0, keepdims=True)
        nec = ne[c * NE:(c + 1) * NE, :]
        rows.append(jnp.sum(nec * wme, axis=0, keepdims=True))
    wsum = jnp.concatenate(rows, axis=0)
    out_ref[...] = jnp.dot(wsum, wproj_ref[...], preferred_element_type=jnp.float32)


def _candnb_pallas(ne_rows, mask_t, W_ge, b_ge, v_ge, W_gproj):
    M = B * C
    return pl.pallas_call(
        _candnb_body,
        grid=(M // NBC,),
        in_specs=[
            pl.BlockSpec((NBC * NE, EPAD), lambda i: (i, 0)),
            pl.BlockSpec((1, NE, NBC), lambda i: (i, 0, 0)),
            pl.BlockSpec((EPAD, EPAD), lambda i: (0, 0)),
            pl.BlockSpec((1, EPAD), lambda i: (0, 0)),
            pl.BlockSpec((1, EPAD), lambda i: (0, 0)),
            pl.BlockSpec((EPAD, D), lambda i: (0, 0)),
        ],
        out_specs=pl.BlockSpec((NBC, D), lambda i: (i, 0)),
        out_shape=jax.ShapeDtypeStruct((M, D), jnp.float32),
    )(ne_rows, mask_t, W_ge, b_ge, v_ge, W_gproj)


def _score_loss_body(cand_ref, user_ref, label_ref, loss_ref, score_ref):
    cand = cand_ref[...]
    user = user_ref[...]
    score = jnp.sum(cand * user[:, None, :], axis=-1)
    m = jnp.max(score, axis=-1, keepdims=True)
    lse = m + jnp.log(jnp.sum(jnp.exp(score - m), axis=-1, keepdims=True))
    logp = score - lse
    lbl = label_ref[...]
    onehot = (lax.broadcasted_iota(jnp.int32, (B, C), 1) == lbl[:, None]).astype(jnp.float32)
    loss = -jnp.mean(jnp.sum(logp * onehot, axis=-1))
    loss_ref[...] = jnp.broadcast_to(loss, (1, 1))
    score_ref[...] = score


def _score_loss(cand_final, user_emb, label):
    loss, score = pl.pallas_call(
        _score_loss_body,
        out_shape=(jax.ShapeDtypeStruct((1, 1), jnp.float32),
                   jax.ShapeDtypeStruct((B, C), jnp.float32)),
    )(cand_final, user_emb, label)
    return loss[0, 0], score


def kernel(subgraph_x, edge_index, mapping_idx, candidate_news, candidate_entity, entity_mask, label,
           glove, entity_table, Wq, Wk, Wv, W_att, b_att, v_att, W_ent, W_ggc,
           W_ih, W_hh, b_ih, b_hh, W_lih, W_lhh, b_l, W_ge, b_ge, v_ge, W_gproj):
    f32 = jnp.float32
    Wqkv = jnp.concatenate([Wq, Wk, Wv], axis=1)
    ent_pad = jnp.pad(entity_table, ((0, 0), (0, EPAD - ED)))
    tokens_all = jnp.concatenate(
        [subgraph_x[:, :NUM_TOK],
         candidate_news[..., :NUM_TOK].reshape(B * C, NUM_TOK)], axis=0)
    tok_pad = jnp.pad(tokens_all, ((0, 0), (0, TPAD - NUM_TOK)))
    ent_ids = jnp.concatenate(
        [subgraph_x[:, -8:-3],
         candidate_entity[..., :ES].reshape(B * C, ES)], axis=0)

    glove_qkv, ent_proj = _prep_pallas(glove, entity_table, Wqkv, W_ent)

    tok_idx3 = tok_pad.reshape(NW, TOK_NCH, TOK_CH)
    ent_idx3 = jnp.pad(ent_ids.reshape(-1),
                       (0, ENTROWS - M_ALL * ES)).reshape(NW, ENT_NCH, ENT_CH)
    neighbor_e = candidate_entity[..., ES:].reshape(B * C * NE)
    nb_idx3 = jnp.pad(neighbor_e, (0, NBROWS_P - NBROWS)).reshape(NW, NBE_NCH, NBE_CH)
    qkv_flat, ent_rows, ne_rows_p = _sc_gather_all(glove_qkv, ent_proj, ent_pad,
                                                   tok_idx3, ent_idx3, nb_idx3)
    ne_rows = ne_rows_p[:NBROWS]
    entm = ent_rows[:M_ALL * ES].reshape(M_ALL, ES, D)

    x_all, m0_all = _encode_pallas(qkv_flat, entm, W_att,
                                   b_att.reshape(1, D), v_att.reshape(1, D),
                                   W_ggc[0])
    x_encoded = x_all[:N_NODES]
    cand_title = x_all[N_NODES:].reshape(B, C, D)

    srcidx3 = edge_index[0].reshape(NW, E_NCH, E_CH)
    dstidx3 = edge_index[1].reshape(NW, E_NCH, E_CH)
    zrows = jnp.zeros((NODES_PER_TILE, D), f32)
    h = x_encoded
    m = m0_all[:N_NODES]
    for l in range(3):
        agg2 = _sc_segsum(m, srcidx3, dstidx3, zrows)
        W_next = W_ggc[l + 1] if l < 2 else jnp.zeros((D, D), f32)
        h, m = _gru_pallas(h, agg2, W_ih, W_hh, b_ih.reshape(1, 3 * D),
                           b_hh.reshape(1, 3 * D), W_next)
    graph_emb = h

    map_idx3 = jnp.pad(mapping_idx.T.reshape(-1),
                       (0, NW * CLK_PT - CLK)).reshape(NW, 1, CLK_PT)
    clk_x, clk_g = _sc_gather_clicked(x_encoded, graph_emb, map_idx3)
    clicked_tm = jnp.concatenate([clk_x, clk_g], axis=-1)
    user_emb = _lstm_pallas(clicked_tm, W_lih, W_lhh, b_l.reshape(1, 4 * UD))

    mask_t = entity_mask.reshape(B * C // NBC, NBC, NE).transpose(0, 2, 1)
    W_ge_pad = jnp.pad(W_ge, ((0, EPAD - ED), (0, EPAD - ED)))
    b_ge_pad = jnp.pad(b_ge, (0, EPAD - ED)).reshape(1, EPAD)
    v_ge_pad = jnp.pad(v_ge, (0, EPAD - ED)).reshape(1, EPAD)
    W_gproj_pad = jnp.pad(W_gproj, ((0, EPAD - ED), (0, 0)))
    cand_nb = _candnb_pallas(ne_rows, mask_t, W_ge_pad, b_ge_pad, v_ge_pad,
                             W_gproj_pad).reshape(B, C, D)

    cand_final = jnp.concatenate([cand_nb, cand_title], axis=-1)
    loss, score = _score_loss(cand_final, user_emb, label)
    return (loss, score)


# speedup vs baseline: 2.4113x; 1.1746x over previous
import functools

import jax
import jax.numpy as jnp
import numpy as np
from jax import lax
from jax.experimental import pallas as pl
from jax.experimental.pallas import tpu as pltpu
from jax.experimental.pallas import tpu_sc as plsc

N_NODES = 10000; N_EDGES = 320000; TOKEN_DIM = 38; NUM_TOK = 30
B = 64; NC = 50; C = 5; ES = 5; EN = 10
GV = 100000; EV = 100000; WD = 300; ED = 100; D = 128; UD = 256; H = 8; HD = 16
TPAD = 32
EPAD = 128
QKVW = 3 * D
NB = 16
NE = ES * EN

NSC = 2; NTILE = 16; NW = NSC * NTILE
M_ALL = N_NODES + B * C
TOKROWS = M_ALL * TPAD
TOK_NCH, TOK_CH = 258, 40
ENTROWS = 53760
ENT_NCH, ENT_CH = 14, 120
NBROWS = B * C * NE
NBROWS_P = 16384
NBE_NCH, NBE_CH = 4, 128
E_NCH, E_CH = 125, 80
ACC_ROWS = 10240
NODES_PER_TILE = ACC_ROWS // NTILE
PREP_BLK = 400


def _prep_body(g_ref, e_ref, wqkv_ref, went_ref, qkv_ref, entp_ref):
    went = went_ref[0:ED] + went_ref[ED:2 * ED] + went_ref[2 * ED:3 * ED]
    qkv_ref[...] = jnp.dot(g_ref[...], wqkv_ref[...],
                           preferred_element_type=jnp.float32)
    entp_ref[...] = jnp.dot(e_ref[...], went,
                            preferred_element_type=jnp.float32)


def _prep_pallas(glove, entity_table, Wqkv, W_ent):
    return pl.pallas_call(
        _prep_body,
        grid=(GV // PREP_BLK,),
        in_specs=[
            pl.BlockSpec((PREP_BLK, WD), lambda i: (i, 0)),
            pl.BlockSpec((PREP_BLK, ED), lambda i: (i, 0)),
            pl.BlockSpec((WD, QKVW), lambda i: (0, 0)),
            pl.BlockSpec((3 * ED, D), lambda i: (0, 0)),
        ],
        out_specs=[
            pl.BlockSpec((PREP_BLK, QKVW), lambda i: (i, 0)),
            pl.BlockSpec((PREP_BLK, D), lambda i: (i, 0)),
        ],
        out_shape=[
            jax.ShapeDtypeStruct((GV, QKVW), jnp.float32),
            jax.ShapeDtypeStruct((EV, D), jnp.float32),
        ],
    )(glove, entity_table, Wqkv, W_ent)


def _sc_gather_all(glove_qkv, ent_proj, ent_pad, tok_idx3, ent_idx3, nb_idx3):
    mesh = plsc.VectorSubcoreMesh(core_axis_name="c", subcore_axis_name="s")

    @functools.partial(
        pl.kernel,
        out_type=[jax.ShapeDtypeStruct((TOKROWS, QKVW), jnp.float32),
                  jax.ShapeDtypeStruct((ENTROWS, D), jnp.float32),
                  jax.ShapeDtypeStruct((NBROWS_P, EPAD), jnp.float32)],
        mesh=mesh,
        scratch_types=[pltpu.VMEM((TOK_NCH, TOK_CH), jnp.int32),
                       pltpu.VMEM((ENT_NCH, ENT_CH), jnp.int32),
                       pltpu.VMEM((NBE_NCH, NBE_CH), jnp.int32),
                       pltpu.VMEM((TOK_CH, QKVW), jnp.float32),
                       pltpu.VMEM((TOK_CH, QKVW), jnp.float32),
                       pltpu.VMEM((ENT_CH, D), jnp.float32),
                       pltpu.VMEM((ENT_CH, D), jnp.float32),
                       pltpu.VMEM((NBE_CH, EPAD), jnp.float32),
                       pltpu.SemaphoreType.DMA,
                       pltpu.SemaphoreType.DMA,
                       pltpu.SemaphoreType.DMA,
                       pltpu.SemaphoreType.DMA],
    )
    def k(gq_hbm, ep_hbm, er_hbm, tokidx_hbm, entidx_hbm, nbidx_hbm,
          qkvout_hbm, entout_hbm, nbout_hbm,
          tokidx_v, entidx_v, nbidx_v, tokbuf_a, tokbuf_b, entbuf_a, entbuf_b,
          nbbuf, sem_a, sem_b, sem_wa, sem_wb):
        cid = lax.axis_index("c"); sid = lax.axis_index("s")
        wid = sid * NSC + cid
        pltpu.sync_copy(tokidx_hbm.at[wid], tokidx_v)
        pltpu.sync_copy(entidx_hbm.at[wid], entidx_v)
        pltpu.sync_copy(nbidx_hbm.at[wid], nbidx_v)
        tbase = wid * (TOK_NCH * TOK_CH)

        @pl.loop(0, TOK_NCH, step=2)
        def _tok(j):
            ga = pltpu.async_copy(gq_hbm.at[tokidx_v.at[j]], tokbuf_a, sem_a)
            gb = pltpu.async_copy(gq_hbm.at[tokidx_v.at[j + 1]], tokbuf_b, sem_b)
            ga.wait()
            wa = pltpu.async_copy(
                tokbuf_a, qkvout_hbm.at[pl.ds(tbase + j * TOK_CH, TOK_CH)], sem_wa)
            gb.wait()
            wb = pltpu.async_copy(
                tokbuf_b, qkvout_hbm.at[pl.ds(tbase + (j + 1) * TOK_CH, TOK_CH)], sem_wb)
            wa.wait()
            wb.wait()

        ebase = wid * (ENT_NCH * ENT_CH)

        @pl.loop(0, ENT_NCH, step=2)
        def _ent(j):
            ga = pltpu.async_copy(ep_hbm.at[entidx_v.at[j]], entbuf_a, sem_a)
            gb = pltpu.async_copy(ep_hbm.at[entidx_v.at[j + 1]], entbuf_b, sem_b)
            ga.wait()
            wa = pltpu.async_copy(
                entbuf_a, entout_hbm.at[pl.ds(ebase + j * ENT_CH, ENT_CH)], sem_wa)
            gb.wait()
            wb = pltpu.async_copy(
                entbuf_b, entout_hbm.at[pl.ds(ebase + (j + 1) * ENT_CH, ENT_CH)], sem_wb)
            wa.wait()
            wb.wait()

        nbase = wid * (NBE_NCH * NBE_CH)

        @pl.loop(0, NBE_NCH)
        def _nb(j):
            pltpu.async_copy(er_hbm.at[nbidx_v.at[j]], nbbuf, sem_a).wait()
            pltpu.sync_copy(nbbuf, nbout_hbm.at[pl.ds(nbase + j * NBE_CH, NBE_CH)])

    return k(glove_qkv, ent_proj, ent_pad, tok_idx3, ent_idx3, nb_idx3)


def _sc_segsum(m, srcidx3, dstidx3, zrows):
    mesh = plsc.VectorSubcoreMesh(core_axis_name="c", subcore_axis_name="s")

    @functools.partial(
        pl.kernel,
        out_type=jax.ShapeDtypeStruct((NSC, ACC_ROWS, D), jnp.float32),
        mesh=mesh,
        scratch_types=[pltpu.VMEM((E_NCH, E_CH), jnp.int32),
                       pltpu.VMEM((E_NCH, E_CH), jnp.int32),
                       pltpu.VMEM((E_CH, D), jnp.float32),
                       pltpu.VMEM_SHARED((ACC_ROWS, D), jnp.float32),
                       pltpu.SemaphoreType.DMA],
    )
    def k(m_hbm, srcidx_hbm, dstidx_hbm, z_hbm, out_hbm,
          sidx_v, didx_v, rows_v, acc, sem):
        cid = lax.axis_index("c"); sid = lax.axis_index("s")
        wid2 = cid * NTILE + sid
        pltpu.sync_copy(srcidx_hbm.at[wid2], sidx_v)
        pltpu.sync_copy(dstidx_hbm.at[wid2], didx_v)
        pltpu.sync_copy(z_hbm, acc.at[pl.ds(sid * NODES_PER_TILE, NODES_PER_TILE)])
        plsc.subcore_barrier()

        @pl.loop(0, E_NCH)
        def _e(j):
            pltpu.async_copy(m_hbm.at[sidx_v.at[j]], rows_v, sem).wait()
            pltpu.sync_copy(rows_v, acc.at[didx_v.at[j]], add=True)

        plsc.subcore_barrier()
        pltpu.sync_copy(
            acc.at[pl.ds(sid * NODES_PER_TILE, NODES_PER_TILE)],
            out_hbm.at[cid].at[pl.ds(sid * NODES_PER_TILE, NODES_PER_TILE)])

    return k(m, srcidx3, dstidx3, zrows)


CLK = NC * B
CLK_PT = 128
CLK_TILES = CLK // CLK_PT


def _sc_gather_clicked(xenc, gemb, map_idx3):
    mesh = plsc.VectorSubcoreMesh(core_axis_name="c", subcore_axis_name="s")

    @functools.partial(
        pl.kernel,
        out_type=[jax.ShapeDtypeStruct((CLK, D), jnp.float32),
                  jax.ShapeDtypeStruct((CLK, D), jnp.float32)],
        mesh=mesh,
        scratch_types=[pltpu.VMEM((1, CLK_PT), jnp.int32),
                       pltpu.VMEM((CLK_PT, D), jnp.float32),
                       pltpu.SemaphoreType.DMA],
    )
    def k(xenc_hbm, gemb_hbm, mapidx_hbm, out1_hbm, out2_hbm, idx_v, buf, sem):
        cid = lax.axis_index("c"); sid = lax.axis_index("s")
        wid = sid * NSC + cid

        @pl.when(wid < CLK_TILES)
        def _():
            pltpu.sync_copy(mapidx_hbm.at[wid], idx_v)
            pltpu.async_copy(xenc_hbm.at[idx_v.at[0]], buf, sem).wait()
            pltpu.sync_copy(buf, out1_hbm.at[pl.ds(wid * CLK_PT, CLK_PT)])
            pltpu.async_copy(gemb_hbm.at[idx_v.at[0]], buf, sem).wait()
            pltpu.sync_copy(buf, out2_hbm.at[pl.ds(wid * CLK_PT, CLK_PT)])

    return k(xenc, gemb, map_idx3)


def _encode_body(qkv_ref, entm_ref, watt_ref, batt_ref, vatt_ref,
                 wggc0_ref, xenc_ref, m0_ref, es_scr, out_scr):
    bf16 = jnp.bfloat16
    qkv = qkv_ref[...]
    rowmod = lax.broadcasted_iota(jnp.int32, (NB * TPAD, QKVW), 0) % TPAD
    qkv = jnp.where(rowmod < NUM_TOK, qkv, 0.0)

    i0 = lax.broadcasted_iota(jnp.int32, (4 * TPAD, D), 0)
    i1 = lax.broadcasted_iota(jnp.int32, (4 * TPAD, D), 1)
    hm0 = (i0 // TPAD == i1 // HD).astype(bf16)
    hm1 = (i0 // TPAD == i1 // HD - 4).astype(bf16)
    o0 = lax.broadcasted_iota(jnp.int32, (H * TPAD, H), 0)
    o1 = lax.broadcasted_iota(jnp.int32, (H * TPAD, H), 1)
    onesbd = ((o0 // TPAD == o1) & (o0 % TPAD < NUM_TOK)).astype(jnp.float32)
    x0 = lax.broadcasted_iota(jnp.int32, (H, D), 0)
    x1 = lax.broadcasted_iota(jnp.int32, (H, D), 1)
    expand = (x0 == x1 // HD).astype(jnp.float32)
    tmask = lax.broadcasted_iota(jnp.int32, (TPAD, 1), 0) < NUM_TOK

    nt = (((1,), (1,)), ((), ()))
    nn = (((1,), (0,)), ((), ()))
    for n in range(NB):
        q = qkv[n * TPAD:(n + 1) * TPAD, 0:D].astype(bf16)
        k = qkv[n * TPAD:(n + 1) * TPAD, D:2 * D].astype(bf16)
        v = qkv[n * TPAD:(n + 1) * TPAD, 2 * D:3 * D].astype(bf16)
        k4 = jnp.concatenate([k] * 4, axis=0)
        v4 = jnp.concatenate([v] * 4, axis=0)
        s0 = lax.dot_general(q, k4 * hm0, nt, preferred_element_type=jnp.float32)
        s1 = lax.dot_general(q, k4 * hm1, nt, preferred_element_type=jnp.float32)
        scores = jnp.concatenate([s0, s1], axis=1) * 0.25
        mrow = jnp.max(scores, axis=1, keepdims=True)
        es = jnp.exp(scores - mrow)
        es_scr[pl.ds(n * TPAD, TPAD), :] = es
        esb = es.astype(bf16)
        ou = (lax.dot_general(esb[:, 0:D], v4 * hm0, nn,
                              preferred_element_type=jnp.float32)
              + lax.dot_general(esb[:, D:2 * D], v4 * hm1, nn,
                                preferred_element_type=jnp.float32))
        out_scr[pl.ds(n * TPAD, TPAD), :] = ou

    es_all = es_scr[...]
    denom = lax.dot_general(es_all, onesbd, nn,
                            preferred_element_type=jnp.float32)
    recipb = lax.dot_general(1.0 / denom, expand, nn,
                             preferred_element_type=jnp.float32)
    out_all = out_scr[...] * recipb
    an = jnp.tanh(jnp.dot(out_all, watt_ref[...],
                          preferred_element_type=jnp.float32) + batt_ref[...])
    alv = jnp.sum(an * vatt_ref[...], axis=1, keepdims=True)
    pooled_rows = []
    for n in range(NB):
        al = jnp.where(tmask, alv[n * TPAD:(n + 1) * TPAD, :], -1e30)
        wm = jnp.exp(al - jnp.max(al, axis=0, keepdims=True))
        wm = wm / jnp.sum(wm, axis=0, keepdims=True)
        outn = out_all[n * TPAD:(n + 1) * TPAD, :]
        pooled_rows.append(jnp.sum(outn * wm, axis=0, keepdims=True))
    pooledb = jnp.concatenate(pooled_rows, axis=0)
    entm = jnp.mean(entm_ref[...], axis=1)
    xe = pooledb + entm
    xenc_ref[...] = xe
    m0_ref[...] = jnp.dot(xe, wggc0_ref[...], preferred_element_type=jnp.float32)


def _encode_pallas(qkv_flat, entm, W_att, b_att, v_att, W_ggc0):
    M = entm.shape[0]
    grid = M // NB
    return pl.pallas_call(
        _encode_body,
        grid=(grid,),
        in_specs=[
            pl.BlockSpec((NB * TPAD, QKVW), lambda i: (i, 0)),
            pl.BlockSpec((NB, ES, D), lambda i: (i, 0, 0)),
            pl.BlockSpec((D, D), lambda i: (0, 0)),
            pl.BlockSpec((1, D), lambda i: (0, 0)),
            pl.BlockSpec((1, D), lambda i: (0, 0)),
            pl.BlockSpec((D, D), lambda i: (0, 0)),
        ],
        out_specs=[
            pl.BlockSpec((NB, D), lambda i: (i, 0)),
            pl.BlockSpec((NB, D), lambda i: (i, 0)),
        ],
        out_shape=[
            jax.ShapeDtypeStruct((M, D), jnp.float32),
            jax.ShapeDtypeStruct((M, D), jnp.float32),
        ],
        scratch_shapes=[
            pltpu.VMEM((NB * TPAD, H * TPAD), jnp.float32),
            pltpu.VMEM((NB * TPAD, D), jnp.float32),
        ],
    )(qkv_flat, entm, W_att, b_att, v_att, W_ggc0)


GRU_BLK = 400


def _gru_body(h_ref, agg0_ref, agg1_ref, wih_ref, whh_ref, bih_ref, bhh_ref, wnext_ref,
              hout_ref, mnext_ref):
    h = h_ref[...]
    agg = agg0_ref[0] + agg1_ref[0]
    gi = jnp.dot(agg, wih_ref[...], preferred_element_type=jnp.float32) + bih_ref[...]
    gh = jnp.dot(h, whh_ref[...], preferred_element_type=jnp.float32) + bhh_ref[...]
    r = jax.nn.sigmoid(gi[:, :D] + gh[:, :D])
    z = jax.nn.sigmoid(gi[:, D:2 * D] + gh[:, D:2 * D])
    n = jnp.tanh(gi[:, 2 * D:] + r * gh[:, 2 * D:])
    hn = (1.0 - z) * n + z * h
    hout_ref[...] = hn
    mnext_ref[...] = jnp.dot(hn, wnext_ref[...], preferred_element_type=jnp.float32)


def _gru_pallas(h, agg2, W_ih, W_hh, b_ih, b_hh, W_next):
    M = h.shape[0]
    return pl.pallas_call(
        _gru_body,
        grid=(M // GRU_BLK,),
        in_specs=[
            pl.BlockSpec((GRU_BLK, D), lambda i: (i, 0)),
            pl.BlockSpec((1, GRU_BLK, D), lambda i: (0, i, 0)),
            pl.BlockSpec((1, GRU_BLK, D), lambda i: (1, i, 0)),
            pl.BlockSpec((D, 3 * D), lambda i: (0, 0)),
            pl.BlockSpec((D, 3 * D), lambda i: (0, 0)),
            pl.BlockSpec((1, 3 * D), lambda i: (0, 0)),
            pl.BlockSpec((1, 3 * D), lambda i: (0, 0)),
            pl.BlockSpec((D, D), lambda i: (0, 0)),
        ],
        out_specs=[
            pl.BlockSpec((GRU_BLK, D), lambda i: (i, 0)),
            pl.BlockSpec((GRU_BLK, D), lambda i: (i, 0)),
        ],
        out_shape=[
            jax.ShapeDtypeStruct((M, D), jnp.float32),
            jax.ShapeDtypeStruct((M, D), jnp.float32),
        ],
    )(h, agg2, agg2, W_ih, W_hh, b_ih, b_hh, W_next)


def _lstm_body(x_ref, wih_ref, whh_ref, b_ref, hout_ref):
    wih = wih_ref[...]; whh = whh_ref[...]; bb = b_ref[...]

    def step(t, carry):
        hs, cs = carry
        xt = x_ref[pl.ds(t * B, B), :]
        g = (jnp.dot(xt, wih, preferred_element_type=jnp.float32)
             + jnp.dot(hs, whh, preferred_element_type=jnp.float32) + bb)
        i = jax.nn.sigmoid(g[:, :UD])
        f = jax.nn.sigmoid(g[:, UD:2 * UD])
        gg = jnp.tanh(g[:, 2 * UD:3 * UD])
        o = jax.nn.sigmoid(g[:, 3 * UD:])
        cs = f * cs + i * gg
        hs = o * jnp.tanh(cs)
        return (hs, cs)

    z = jnp.zeros((B, UD), jnp.float32)
    hs, _ = lax.fori_loop(0, NC, step, (z, z))
    hout_ref[...] = hs


def _lstm_pallas(clicked_tm, W_lih, W_lhh, b_l):
    return pl.pallas_call(
        _lstm_body,
        out_shape=jax.ShapeDtypeStruct((B, UD), jnp.float32),
    )(clicked_tm, W_lih, W_lhh, b_l)


NBC = 8


def _candnb_body(ne_ref, maskt_ref, wge_ref, bge_ref, vge_ref, wproj_ref, out_ref):
    ne = ne_ref[...]
    t = jnp.tanh(jnp.dot(ne, wge_ref[...], preferred_element_type=jnp.float32)
                 + bge_ref[...])
    a = jnp.sum(t * vge_ref[...], axis=1, keepdims=True)
    rows = []
    for c in range(NBC):
        ac = a[c * NE:(c + 1) * NE, :]
        mc = maskt_ref[0, :, c:c + 1]
        ac = jnp.where(mc > 0, ac, -1e9)
        wme = jnp.exp(ac - jnp.max(ac, axis=0, keepdims=True))
        wme = wme / jnp.sum(wme, axis=0, keepdims=True)
        nec = ne[c * NE:(c + 1) * NE, :]
        rows.append(jnp.sum(nec * wme, axis=0, keepdims=True))
    wsum = jnp.concatenate(rows, axis=0)
    out_ref[...] = jnp.dot(wsum, wproj_ref[...], preferred_element_type=jnp.float32)


def _candnb_pallas(ne_rows, mask_t, W_ge, b_ge, v_ge, W_gproj):
    M = B * C
    return pl.pallas_call(
        _candnb_body,
        grid=(M // NBC,),
        in_specs=[
            pl.BlockSpec((NBC * NE, EPAD), lambda i: (i, 0)),
            pl.BlockSpec((1, NE, NBC), lambda i: (i, 0, 0)),
            pl.BlockSpec((EPAD, EPAD), lambda i: (0, 0)),
            pl.BlockSpec((1, EPAD), lambda i: (0, 0)),
            pl.BlockSpec((1, EPAD), lambda i: (0, 0)),
            pl.BlockSpec((EPAD, D), lambda i: (0, 0)),
        ],
        out_specs=pl.BlockSpec((NBC, D), lambda i: (i, 0)),
        out_shape=jax.ShapeDtypeStruct((M, D), jnp.float32),
    )(ne_rows, mask_t, W_ge, b_ge, v_ge, W_gproj)


def _score_loss_body(cand_ref, user_ref, label_ref, loss_ref, score_ref):
    cand = cand_ref[...]
    user = user_ref[...]
    score = jnp.sum(cand * user[:, None, :], axis=-1)
    m = jnp.max(score, axis=-1, keepdims=True)
    lse = m + jnp.log(jnp.sum(jnp.exp(score - m), axis=-1, keepdims=True))
    logp = score - lse
    lbl = label_ref[...]
    onehot = (lax.broadcasted_iota(jnp.int32, (B, C), 1) == lbl[:, None]).astype(jnp.float32)
    loss = -jnp.mean(jnp.sum(logp * onehot, axis=-1))
    loss_ref[...] = jnp.broadcast_to(loss, (1, 1))
    score_ref[...] = score


def _score_loss(cand_final, user_emb, label):
    loss, score = pl.pallas_call(
        _score_loss_body,
        out_shape=(jax.ShapeDtypeStruct((1, 1), jnp.float32),
                   jax.ShapeDtypeStruct((B, C), jnp.float32)),
    )(cand_final, user_emb, label)
    return loss[0, 0], score


def kernel(subgraph_x, edge_index, mapping_idx, candidate_news, candidate_entity, entity_mask, label,
           glove, entity_table, Wq, Wk, Wv, W_att, b_att, v_att, W_ent, W_ggc,
           W_ih, W_hh, b_ih, b_hh, W_lih, W_lhh, b_l, W_ge, b_ge, v_ge, W_gproj):
    f32 = jnp.float32
    Wqkv = jnp.concatenate([Wq, Wk, Wv], axis=1)
    ent_pad = jnp.pad(entity_table, ((0, 0), (0, EPAD - ED)))
    tokens_all = jnp.concatenate(
        [subgraph_x[:, :NUM_TOK],
         candidate_news[..., :NUM_TOK].reshape(B * C, NUM_TOK)], axis=0)
    tok_pad = jnp.pad(tokens_all, ((0, 0), (0, TPAD - NUM_TOK)))
    ent_ids = jnp.concatenate(
        [subgraph_x[:, -8:-3],
         candidate_entity[..., :ES].reshape(B * C, ES)], axis=0)

    glove_qkv, ent_proj = _prep_pallas(glove, entity_table, Wqkv, W_ent)

    tok_idx3 = tok_pad.reshape(NW, TOK_NCH, TOK_CH)
    ent_idx3 = jnp.pad(ent_ids.reshape(-1),
                       (0, ENTROWS - M_ALL * ES)).reshape(NW, ENT_NCH, ENT_CH)
    neighbor_e = candidate_entity[..., ES:].reshape(B * C * NE)
    nb_idx3 = jnp.pad(neighbor_e, (0, NBROWS_P - NBROWS)).reshape(NW, NBE_NCH, NBE_CH)
    qkv_flat, ent_rows, ne_rows_p = _sc_gather_all(glove_qkv, ent_proj, ent_pad,
                                                   tok_idx3, ent_idx3, nb_idx3)
    ne_rows = ne_rows_p[:NBROWS]
    entm = ent_rows[:M_ALL * ES].reshape(M_ALL, ES, D)

    x_all, m0_all = _encode_pallas(qkv_flat, entm, W_att,
                                   b_att.reshape(1, D), v_att.reshape(1, D),
                                   W_ggc[0])
    x_encoded = x_all[:N_NODES]
    cand_title = x_all[N_NODES:].reshape(B, C, D)

    srcidx3 = edge_index[0].reshape(NW, E_NCH, E_CH)
    dstidx3 = edge_index[1].reshape(NW, E_NCH, E_CH)
    zrows = jnp.zeros((NODES_PER_TILE, D), f32)
    h = x_encoded
    m = m0_all[:N_NODES]
    for l in range(3):
        agg2 = _sc_segsum(m, srcidx3, dstidx3, zrows)
        W_next = W_ggc[l + 1] if l < 2 else jnp.zeros((D, D), f32)
        h, m = _gru_pallas(h, agg2, W_ih, W_hh, b_ih.reshape(1, 3 * D),
                           b_hh.reshape(1, 3 * D), W_next)
    graph_emb = h

    map_idx3 = jnp.pad(mapping_idx.T.reshape(-1),
                       (0, NW * CLK_PT - CLK)).reshape(NW, 1, CLK_PT)
    clk_x, clk_g = _sc_gather_clicked(x_encoded, graph_emb, map_idx3)
    clicked_tm = jnp.concatenate([clk_x, clk_g], axis=-1)
    user_emb = _lstm_pallas(clicked_tm, W_lih, W_lhh, b_l.reshape(1, 4 * UD))

    mask_t = entity_mask.reshape(B * C // NBC, NBC, NE).transpose(0, 2, 1)
    W_ge_pad = jnp.pad(W_ge, ((0, EPAD - ED), (0, EPAD - ED)))
    b_ge_pad = jnp.pad(b_ge, (0, EPAD - ED)).reshape(1, EPAD)
    v_ge_pad = jnp.pad(v_ge, (0, EPAD - ED)).reshape(1, EPAD)
    W_gproj_pad = jnp.pad(W_gproj, ((0, EPAD - ED), (0, 0)))
    cand_nb = _candnb_pallas(ne_rows, mask_t, W_ge_pad, b_ge_pad, v_ge_pad,
                             W_gproj_pad).reshape(B, C, D)

    cand_final = jnp.concatenate([cand_nb, cand_title], axis=-1)
    loss, score = _score_loss(cand_final, user_emb, label)
    return (loss, score)


# speedup vs baseline: 3.5512x; 1.4727x over previous
import functools

import jax
import jax.numpy as jnp
import numpy as np
from jax import lax
from jax.experimental import pallas as pl
from jax.experimental.pallas import tpu as pltpu
from jax.experimental.pallas import tpu_sc as plsc

N_NODES = 10000; N_EDGES = 320000; TOKEN_DIM = 38; NUM_TOK = 30
B = 64; NC = 50; C = 5; ES = 5; EN = 10
GV = 100000; EV = 100000; WD = 300; ED = 100; D = 128; UD = 256; H = 8; HD = 16
TPAD = 32
EPAD = 128
QKVW = 3 * D
NB = 16
NE = ES * EN

NSC = 2; NTILE = 16; NW = NSC * NTILE
M_ALL = N_NODES + B * C
TOKROWS = M_ALL * TPAD
TOK_NCH, TOK_CH = 430, 24
ENTROWS = 53760
ENT_NCH, ENT_CH = 14, 120
NBROWS = B * C * NE
NBROWS_P = 16384
NBE_NCH, NBE_CH = 8, 64
E_NCH, E_CH = 125, 80
ACC_ROWS = 10240
NODES_PER_TILE = ACC_ROWS // NTILE
PREP_BLK = 400


def _prep_body(g_ref, e_ref, wqkv_ref, went_ref, qkv_ref, entp_ref):
    went = went_ref[0:ED] + went_ref[ED:2 * ED] + went_ref[2 * ED:3 * ED]
    qkv_ref[...] = jnp.dot(g_ref[...], wqkv_ref[...],
                           preferred_element_type=jnp.float32)
    entp_ref[...] = jnp.dot(e_ref[...], went,
                            preferred_element_type=jnp.float32)


def _prep_pallas(glove, entity_table, Wqkv, W_ent):
    return pl.pallas_call(
        _prep_body,
        grid=(GV // PREP_BLK,),
        in_specs=[
            pl.BlockSpec((PREP_BLK, WD), lambda i: (i, 0)),
            pl.BlockSpec((PREP_BLK, ED), lambda i: (i, 0)),
            pl.BlockSpec((WD, QKVW), lambda i: (0, 0)),
            pl.BlockSpec((3 * ED, D), lambda i: (0, 0)),
        ],
        out_specs=[
            pl.BlockSpec((PREP_BLK, QKVW), lambda i: (i, 0)),
            pl.BlockSpec((PREP_BLK, D), lambda i: (i, 0)),
        ],
        out_shape=[
            jax.ShapeDtypeStruct((GV, QKVW), jnp.float32),
            jax.ShapeDtypeStruct((EV, D), jnp.float32),
        ],
    )(glove, entity_table, Wqkv, W_ent)


def _sc_gather_all(glove_qkv, ent_proj, ent_pad, tok_idx3, ent_idx3, nb_idx3):
    mesh = plsc.VectorSubcoreMesh(core_axis_name="c", subcore_axis_name="s")

    @functools.partial(
        pl.kernel,
        out_type=[jax.ShapeDtypeStruct((TOKROWS, QKVW), jnp.float32),
                  jax.ShapeDtypeStruct((ENTROWS, D), jnp.float32),
                  jax.ShapeDtypeStruct((NBROWS_P, EPAD), jnp.float32)],
        mesh=mesh,
        scratch_types=[pltpu.VMEM((TOK_NCH, TOK_CH), jnp.int32),
                       pltpu.VMEM((ENT_NCH, ENT_CH), jnp.int32),
                       pltpu.VMEM((NBE_NCH, NBE_CH), jnp.int32),
                       pltpu.VMEM((4, TOK_CH, QKVW), jnp.float32),
                       pltpu.VMEM((ENT_CH, D), jnp.float32),
                       pltpu.VMEM((NBE_CH, EPAD), jnp.float32),
                       pltpu.SemaphoreType.DMA,
                       pltpu.SemaphoreType.DMA,
                       pltpu.SemaphoreType.DMA,
                       pltpu.SemaphoreType.DMA,
                       pltpu.SemaphoreType.DMA,
                       pltpu.SemaphoreType.DMA,
                       pltpu.SemaphoreType.DMA,
                       pltpu.SemaphoreType.DMA],
    )
    def k(gq_hbm, ep_hbm, er_hbm, tokidx_hbm, entidx_hbm, nbidx_hbm,
          qkvout_hbm, entout_hbm, nbout_hbm,
          tokidx_v, entidx_v, nbidx_v, tokbuf4, entbuf_a,
          nbbuf, sem_0, sem_1, sem_2, sem_3, sem_w0, sem_w1, sem_w2, sem_w3):
        cid = lax.axis_index("c"); sid = lax.axis_index("s")
        wid = sid * NSC + cid
        pltpu.sync_copy(tokidx_hbm.at[wid], tokidx_v)
        pltpu.sync_copy(entidx_hbm.at[wid], entidx_v)
        pltpu.sync_copy(nbidx_hbm.at[wid], nbidx_v)
        tbase = wid * (TOK_NCH * TOK_CH)
        gsems = (sem_0, sem_1, sem_2, sem_3)
        wsems = (sem_w0, sem_w1, sem_w2, sem_w3)

        @pl.loop(0, TOK_NCH - 2, step=4)
        def _tok(j):
            gds = [pltpu.async_copy(gq_hbm.at[tokidx_v.at[j + i]],
                                    tokbuf4.at[i], gsems[i]) for i in range(4)]
            wds = []
            for i in range(4):
                gds[i].wait()
                wds.append(pltpu.async_copy(
                    tokbuf4.at[i],
                    qkvout_hbm.at[pl.ds(tbase + (j + i) * TOK_CH, TOK_CH)],
                    wsems[i]))
            for i in range(4):
                wds[i].wait()

        @pl.loop(TOK_NCH - 2, TOK_NCH)
        def _tok_tail(j):
            pltpu.async_copy(gq_hbm.at[tokidx_v.at[j]], tokbuf4.at[0], sem_0).wait()
            pltpu.sync_copy(tokbuf4.at[0],
                            qkvout_hbm.at[pl.ds(tbase + j * TOK_CH, TOK_CH)])

        ebase = wid * (ENT_NCH * ENT_CH)

        @pl.loop(0, ENT_NCH)
        def _ent(j):
            pltpu.async_copy(ep_hbm.at[entidx_v.at[j]], entbuf_a, sem_0).wait()
            pltpu.sync_copy(entbuf_a, entout_hbm.at[pl.ds(ebase + j * ENT_CH, ENT_CH)])

        nbase = wid * (NBE_NCH * NBE_CH)

        @pl.loop(0, NBE_NCH)
        def _nb(j):
            pltpu.async_copy(er_hbm.at[nbidx_v.at[j]], nbbuf, sem_0).wait()
            pltpu.sync_copy(nbbuf, nbout_hbm.at[pl.ds(nbase + j * NBE_CH, NBE_CH)])

    return k(glove_qkv, ent_proj, ent_pad, tok_idx3, ent_idx3, nb_idx3)


def _sc_segsum(m, srcidx3, dstidx3, zrows):
    mesh = plsc.VectorSubcoreMesh(core_axis_name="c", subcore_axis_name="s")

    @functools.partial(
        pl.kernel,
        out_type=jax.ShapeDtypeStruct((NSC, ACC_ROWS, D), jnp.float32),
        mesh=mesh,
        scratch_types=[pltpu.VMEM((E_NCH, E_CH), jnp.int32),
                       pltpu.VMEM((E_NCH, E_CH), jnp.int32),
                       pltpu.VMEM((E_CH, D), jnp.float32),
                       pltpu.VMEM_SHARED((ACC_ROWS, D), jnp.float32),
                       pltpu.SemaphoreType.DMA],
    )
    def k(m_hbm, srcidx_hbm, dstidx_hbm, z_hbm, out_hbm,
          sidx_v, didx_v, rows_v, acc, sem):
        cid = lax.axis_index("c"); sid = lax.axis_index("s")
        wid2 = cid * NTILE + sid
        pltpu.sync_copy(srcidx_hbm.at[wid2], sidx_v)
        pltpu.sync_copy(dstidx_hbm.at[wid2], didx_v)
        pltpu.sync_copy(z_hbm, acc.at[pl.ds(sid * NODES_PER_TILE, NODES_PER_TILE)])
        plsc.subcore_barrier()

        @pl.loop(0, E_NCH)
        def _e(j):
            pltpu.async_copy(m_hbm.at[sidx_v.at[j]], rows_v, sem).wait()
            pltpu.sync_copy(rows_v, acc.at[didx_v.at[j]], add=True)

        plsc.subcore_barrier()
        pltpu.sync_copy(
            acc.at[pl.ds(sid * NODES_PER_TILE, NODES_PER_TILE)],
            out_hbm.at[cid].at[pl.ds(sid * NODES_PER_TILE, NODES_PER_TILE)])

    return k(m, srcidx3, dstidx3, zrows)


CLK = NC * B
CLK_PT = 128
CLK_TILES = CLK // CLK_PT


def _sc_gather_clicked(xenc, gemb, map_idx3):
    mesh = plsc.VectorSubcoreMesh(core_axis_name="c", subcore_axis_name="s")

    @functools.partial(
        pl.kernel,
        out_type=[jax.ShapeDtypeStruct((CLK, D), jnp.float32),
                  jax.ShapeDtypeStruct((CLK, D), jnp.float32)],
        mesh=mesh,
        scratch_types=[pltpu.VMEM((1, CLK_PT), jnp.int32),
                       pltpu.VMEM((CLK_PT, D), jnp.float32),
                       pltpu.SemaphoreType.DMA],
    )
    def k(xenc_hbm, gemb_hbm, mapidx_hbm, out1_hbm, out2_hbm, idx_v, buf, sem):
        cid = lax.axis_index("c"); sid = lax.axis_index("s")
        wid = sid * NSC + cid

        @pl.when(wid < CLK_TILES)
        def _():
            pltpu.sync_copy(mapidx_hbm.at[wid], idx_v)
            pltpu.async_copy(xenc_hbm.at[idx_v.at[0]], buf, sem).wait()
            pltpu.sync_copy(buf, out1_hbm.at[pl.ds(wid * CLK_PT, CLK_PT)])
            pltpu.async_copy(gemb_hbm.at[idx_v.at[0]], buf, sem).wait()
            pltpu.sync_copy(buf, out2_hbm.at[pl.ds(wid * CLK_PT, CLK_PT)])

    return k(xenc, gemb, map_idx3)


def _encode_body(qkv_ref, entm_ref, watt_ref, batt_ref, vatt_ref,
                 wggc0_ref, xenc_ref, m0_ref, es_scr, out_scr):
    bf16 = jnp.bfloat16
    qkv = qkv_ref[...]
    rowmod = lax.broadcasted_iota(jnp.int32, (NB * TPAD, QKVW), 0) % TPAD
    qkv = jnp.where(rowmod < NUM_TOK, qkv, 0.0)

    i0 = lax.broadcasted_iota(jnp.int32, (4 * TPAD, D), 0)
    i1 = lax.broadcasted_iota(jnp.int32, (4 * TPAD, D), 1)
    hm0 = (i0 // TPAD == i1 // HD).astype(bf16)
    hm1 = (i0 // TPAD == i1 // HD - 4).astype(bf16)
    o0 = lax.broadcasted_iota(jnp.int32, (H * TPAD, H), 0)
    o1 = lax.broadcasted_iota(jnp.int32, (H * TPAD, H), 1)
    onesbd = ((o0 // TPAD == o1) & (o0 % TPAD < NUM_TOK)).astype(jnp.float32)
    x0 = lax.broadcasted_iota(jnp.int32, (H, D), 0)
    x1 = lax.broadcasted_iota(jnp.int32, (H, D), 1)
    expand = (x0 == x1 // HD).astype(jnp.float32)
    tmask = lax.broadcasted_iota(jnp.int32, (TPAD, 1), 0) < NUM_TOK

    nt = (((1,), (1,)), ((), ()))
    nn = (((1,), (0,)), ((), ()))
    qkvb = qkv.astype(bf16)
    for n in range(NB):
        q = qkvb[n * TPAD:(n + 1) * TPAD, 0:D]
        k = qkvb[n * TPAD:(n + 1) * TPAD, D:2 * D]
        k4 = jnp.concatenate([k] * 4, axis=0)
        s0 = lax.dot_general(q, k4 * hm0, nt, preferred_element_type=jnp.float32)
        s1 = lax.dot_general(q, k4 * hm1, nt, preferred_element_type=jnp.float32)
        es_scr[pl.ds(n * TPAD, TPAD), :] = jnp.concatenate([s0, s1], axis=1)
    sc_all = es_scr[...] * 0.25
    mrow = jnp.max(sc_all, axis=1, keepdims=True)
    es_all = jnp.exp(sc_all - mrow)
    es_scr[...] = es_all
    esb_all = es_all.astype(bf16)
    for n in range(NB):
        v = qkvb[n * TPAD:(n + 1) * TPAD, 2 * D:3 * D]
        v4 = jnp.concatenate([v] * 4, axis=0)
        esb = esb_all[n * TPAD:(n + 1) * TPAD, :]
        ou = (lax.dot_general(esb[:, 0:D], v4 * hm0, nn,
                              preferred_element_type=jnp.float32)
              + lax.dot_general(esb[:, D:2 * D], v4 * hm1, nn,
                                preferred_element_type=jnp.float32))
        out_scr[pl.ds(n * TPAD, TPAD), :] = ou

    denom = lax.dot_general(es_scr[...], onesbd, nn,
                            preferred_element_type=jnp.float32)
    recipb = lax.dot_general(1.0 / denom, expand, nn,
                             preferred_element_type=jnp.float32)
    out_all = out_scr[...] * recipb
    an = jnp.tanh(jnp.dot(out_all, watt_ref[...],
                          preferred_element_type=jnp.float32) + batt_ref[...])
    alv = jnp.sum(an * vatt_ref[...], axis=1, keepdims=True)
    pooled_rows = []
    for n in range(NB):
        al = jnp.where(tmask, alv[n * TPAD:(n + 1) * TPAD, :], -1e30)
        wm = jnp.exp(al - jnp.max(al, axis=0, keepdims=True))
        wm = wm / jnp.sum(wm, axis=0, keepdims=True)
        outn = out_all[n * TPAD:(n + 1) * TPAD, :]
        pooled_rows.append(jnp.sum(outn * wm, axis=0, keepdims=True))
    pooledb = jnp.concatenate(pooled_rows, axis=0)
    entm = jnp.mean(entm_ref[...], axis=1)
    xe = pooledb + entm
    xenc_ref[...] = xe
    m0_ref[...] = jnp.dot(xe, wggc0_ref[...], preferred_element_type=jnp.float32)


def _encode_pallas(qkv_flat, entm, W_att, b_att, v_att, W_ggc0):
    M = entm.shape[0]
    grid = M // NB
    return pl.pallas_call(
        _encode_body,
        grid=(grid,),
        in_specs=[
            pl.BlockSpec((NB * TPAD, QKVW), lambda i: (i, 0)),
            pl.BlockSpec((NB, ES, D), lambda i: (i, 0, 0)),
            pl.BlockSpec((D, D), lambda i: (0, 0)),
            pl.BlockSpec((1, D), lambda i: (0, 0)),
            pl.BlockSpec((1, D), lambda i: (0, 0)),
            pl.BlockSpec((D, D), lambda i: (0, 0)),
        ],
        out_specs=[
            pl.BlockSpec((NB, D), lambda i: (i, 0)),
            pl.BlockSpec((NB, D), lambda i: (i, 0)),
        ],
        out_shape=[
            jax.ShapeDtypeStruct((M, D), jnp.float32),
            jax.ShapeDtypeStruct((M, D), jnp.float32),
        ],
        scratch_shapes=[
            pltpu.VMEM((NB * TPAD, H * TPAD), jnp.float32),
            pltpu.VMEM((NB * TPAD, D), jnp.float32),
        ],
    )(qkv_flat, entm, W_att, b_att, v_att, W_ggc0)


GRU_BLK = 400


def _gru_body(h_ref, agg0_ref, agg1_ref, wih_ref, whh_ref, bih_ref, bhh_ref, wnext_ref,
              hout_ref, mnext_ref):
    h = h_ref[...]
    agg = agg0_ref[0] + agg1_ref[0]
    gi = jnp.dot(agg, wih_ref[...], preferred_element_type=jnp.float32) + bih_ref[...]
    gh = jnp.dot(h, whh_ref[...], preferred_element_type=jnp.float32) + bhh_ref[...]
    r = jax.nn.sigmoid(gi[:, :D] + gh[:, :D])
    z = jax.nn.sigmoid(gi[:, D:2 * D] + gh[:, D:2 * D])
    n = jnp.tanh(gi[:, 2 * D:] + r * gh[:, 2 * D:])
    hn = (1.0 - z) * n + z * h
    hout_ref[...] = hn
    mnext_ref[...] = jnp.dot(hn, wnext_ref[...], preferred_element_type=jnp.float32)


def _gru_pallas(h, agg2, W_ih, W_hh, b_ih, b_hh, W_next):
    M = h.shape[0]
    return pl.pallas_call(
        _gru_body,
        grid=(M // GRU_BLK,),
        in_specs=[
            pl.BlockSpec((GRU_BLK, D), lambda i: (i, 0)),
            pl.BlockSpec((1, GRU_BLK, D), lambda i: (0, i, 0)),
            pl.BlockSpec((1, GRU_BLK, D), lambda i: (1, i, 0)),
            pl.BlockSpec((D, 3 * D), lambda i: (0, 0)),
            pl.BlockSpec((D, 3 * D), lambda i: (0, 0)),
            pl.BlockSpec((1, 3 * D), lambda i: (0, 0)),
            pl.BlockSpec((1, 3 * D), lambda i: (0, 0)),
            pl.BlockSpec((D, D), lambda i: (0, 0)),
        ],
        out_specs=[
            pl.BlockSpec((GRU_BLK, D), lambda i: (i, 0)),
            pl.BlockSpec((GRU_BLK, D), lambda i: (i, 0)),
        ],
        out_shape=[
            jax.ShapeDtypeStruct((M, D), jnp.float32),
            jax.ShapeDtypeStruct((M, D), jnp.float32),
        ],
    )(h, agg2, agg2, W_ih, W_hh, b_ih, b_hh, W_next)


def _lstm_body(x_ref, wih_ref, whh_ref, b_ref, hout_ref):
    wih = wih_ref[...]; whh = whh_ref[...]; bb = b_ref[...]

    def step(t, carry):
        hs, cs = carry
        xt = x_ref[pl.ds(t * B, B), :]
        g = (jnp.dot(xt, wih, preferred_element_type=jnp.float32)
             + jnp.dot(hs, whh, preferred_element_type=jnp.float32) + bb)
        i = jax.nn.sigmoid(g[:, :UD])
        f = jax.nn.sigmoid(g[:, UD:2 * UD])
        gg = jnp.tanh(g[:, 2 * UD:3 * UD])
        o = jax.nn.sigmoid(g[:, 3 * UD:])
        cs = f * cs + i * gg
        hs = o * jnp.tanh(cs)
        return (hs, cs)

    z = jnp.zeros((B, UD), jnp.float32)
    hs, _ = lax.fori_loop(0, NC, step, (z, z))
    hout_ref[...] = hs


def _lstm_pallas(clicked_tm, W_lih, W_lhh, b_l):
    return pl.pallas_call(
        _lstm_body,
        out_shape=jax.ShapeDtypeStruct((B, UD), jnp.float32),
    )(clicked_tm, W_lih, W_lhh, b_l)


NBC = 8


def _candnb_body(ne_ref, maskt_ref, wge_ref, bge_ref, vge_ref, wproj_ref, out_ref):
    ne = ne_ref[...]
    t = jnp.tanh(jnp.dot(ne, wge_ref[...], preferred_element_type=jnp.float32)
                 + bge_ref[...])
    a = jnp.sum(t * vge_ref[...], axis=1, keepdims=True)
    rows = []
    for c in range(NBC):
        ac = a[c * NE:(c + 1) * NE, :]
        mc = maskt_ref[0, :, c:c + 1]
        ac = jnp.where(mc > 0, ac, -1e9)
        wme = jnp.exp(ac - jnp.max(ac, axis=0, keepdims=True))
        wme = wme / jnp.sum(wme, axis=0, keepdims=True)
        nec = ne[c * NE:(c + 1) * NE, :]
        rows.append(jnp.sum(nec * wme, axis=0, keepdims=True))
    wsum = jnp.concatenate(rows, axis=0)
    out_ref[...] = jnp.dot(wsum, wproj_ref[...], preferred_element_type=jnp.float32)


def _candnb_pallas(ne_rows, mask_t, W_ge, b_ge, v_ge, W_gproj):
    M = B * C
    return pl.pallas_call(
        _candnb_body,
        grid=(M // NBC,),
        in_specs=[
            pl.BlockSpec((NBC * NE, EPAD), lambda i: (i, 0)),
            pl.BlockSpec((1, NE, NBC), lambda i: (i, 0, 0)),
            pl.BlockSpec((EPAD, EPAD), lambda i: (0, 0)),
            pl.BlockSpec((1, EPAD), lambda i: (0, 0)),
            pl.BlockSpec((1, EPAD), lambda i: (0, 0)),
            pl.BlockSpec((EPAD, D), lambda i: (0, 0)),
        ],
        out_specs=pl.BlockSpec((NBC, D), lambda i: (i, 0)),
        out_shape=jax.ShapeDtypeStruct((M, D), jnp.float32),
    )(ne_rows, mask_t, W_ge, b_ge, v_ge, W_gproj)


def _score_loss_body(cand_ref, user_ref, label_ref, loss_ref, score_ref):
    cand = cand_ref[...]
    user = user_ref[...]
    score = jnp.sum(cand * user[:, None, :], axis=-1)
    m = jnp.max(score, axis=-1, keepdims=True)
    lse = m + jnp.log(jnp.sum(jnp.exp(score - m), axis=-1, keepdims=True))
    logp = score - lse
    lbl = label_ref[...]
    onehot = (lax.broadcasted_iota(jnp.int32, (B, C), 1) == lbl[:, None]).astype(jnp.float32)
    loss = -jnp.mean(jnp.sum(logp * onehot, axis=-1))
    loss_ref[...] = jnp.broadcast_to(loss, (1, 1))
    score_ref[...] = score


def _score_loss(cand_final, user_emb, label):
    loss, score = pl.pallas_call(
        _score_loss_body,
        out_shape=(jax.ShapeDtypeStruct((1, 1), jnp.float32),
                   jax.ShapeDtypeStruct((B, C), jnp.float32)),
    )(cand_final, user_emb, label)
    return loss[0, 0], score


def kernel(subgraph_x, edge_index, mapping_idx, candidate_news, candidate_entity, entity_mask, label,
           glove, entity_table, Wq, Wk, Wv, W_att, b_att, v_att, W_ent, W_ggc,
           W_ih, W_hh, b_ih, b_hh, W_lih, W_lhh, b_l, W_ge, b_ge, v_ge, W_gproj):
    f32 = jnp.float32
    Wqkv = jnp.concatenate([Wq, Wk, Wv], axis=1)
    ent_pad = jnp.pad(entity_table, ((0, 0), (0, EPAD - ED)))
    tokens_all = jnp.concatenate(
        [subgraph_x[:, :NUM_TOK],
         candidate_news[..., :NUM_TOK].reshape(B * C, NUM_TOK)], axis=0)
    tok_pad = jnp.pad(tokens_all, ((0, 0), (0, TPAD - NUM_TOK)))
    ent_ids = jnp.concatenate(
        [subgraph_x[:, -8:-3],
         candidate_entity[..., :ES].reshape(B * C, ES)], axis=0)

    glove_qkv, ent_proj = _prep_pallas(glove, entity_table, Wqkv, W_ent)

    tok_idx3 = tok_pad.reshape(NW, TOK_NCH, TOK_CH)
    ent_idx3 = jnp.pad(ent_ids.reshape(-1),
                       (0, ENTROWS - M_ALL * ES)).reshape(NW, ENT_NCH, ENT_CH)
    neighbor_e = candidate_entity[..., ES:].reshape(B * C * NE)
    nb_idx3 = jnp.pad(neighbor_e, (0, NBROWS_P - NBROWS)).reshape(NW, NBE_NCH, NBE_CH)
    qkv_flat, ent_rows, ne_rows_p = _sc_gather_all(glove_qkv, ent_proj, ent_pad,
                                                   tok_idx3, ent_idx3, nb_idx3)
    ne_rows = ne_rows_p[:NBROWS]
    entm = ent_rows[:M_ALL * ES].reshape(M_ALL, ES, D)

    x_all, m0_all = _encode_pallas(qkv_flat, entm, W_att,
                                   b_att.reshape(1, D), v_att.reshape(1, D),
                                   W_ggc[0])
    x_encoded = x_all[:N_NODES]
    cand_title = x_all[N_NODES:].reshape(B, C, D)

    srcidx3 = edge_index[0].reshape(NW, E_NCH, E_CH)
    dstidx3 = edge_index[1].reshape(NW, E_NCH, E_CH)
    zrows = jnp.zeros((NODES_PER_TILE, D), f32)
    h = x_encoded
    m = m0_all[:N_NODES]
    for l in range(3):
        agg2 = _sc_segsum(m, srcidx3, dstidx3, zrows)
        W_next = W_ggc[l + 1] if l < 2 else jnp.zeros((D, D), f32)
        h, m = _gru_pallas(h, agg2, W_ih, W_hh, b_ih.reshape(1, 3 * D),
                           b_hh.reshape(1, 3 * D), W_next)
    graph_emb = h

    map_idx3 = jnp.pad(mapping_idx.T.reshape(-1),
                       (0, NW * CLK_PT - CLK)).reshape(NW, 1, CLK_PT)
    clk_x, clk_g = _sc_gather_clicked(x_encoded, graph_emb, map_idx3)
    clicked_tm = jnp.concatenate([clk_x, clk_g], axis=-1)
    user_emb = _lstm_pallas(clicked_tm, W_lih, W_lhh, b_l.reshape(1, 4 * UD))

    mask_t = entity_mask.reshape(B * C // NBC, NBC, NE).transpose(0, 2, 1)
    W_ge_pad = jnp.pad(W_ge, ((0, EPAD - ED), (0, EPAD - ED)))
    b_ge_pad = jnp.pad(b_ge, (0, EPAD - ED)).reshape(1, EPAD)
    v_ge_pad = jnp.pad(v_ge, (0, EPAD - ED)).reshape(1, EPAD)
    W_gproj_pad = jnp.pad(W_gproj, ((0, EPAD - ED), (0, 0)))
    cand_nb = _candnb_pallas(ne_rows, mask_t, W_ge_pad, b_ge_pad, v_ge_pad,
                             W_gproj_pad).reshape(B, C, D)

    cand_final = jnp.concatenate([cand_nb, cand_title], axis=-1)
    loss, score = _score_loss(cand_final, user_emb, label)
    return (loss, score)


# speedup vs baseline: 3.5663x; 1.0043x over previous
import functools

import jax
import jax.numpy as jnp
import numpy as np
from jax import lax
from jax.experimental import pallas as pl
from jax.experimental.pallas import tpu as pltpu
from jax.experimental.pallas import tpu_sc as plsc

N_NODES = 10000; N_EDGES = 320000; TOKEN_DIM = 38; NUM_TOK = 30
B = 64; NC = 50; C = 5; ES = 5; EN = 10
GV = 100000; EV = 100000; WD = 300; ED = 100; D = 128; UD = 256; H = 8; HD = 16
TPAD = 32
EPAD = 128
QKVW = 3 * D
NB = 16
NE = ES * EN

NSC = 2; NTILE = 16; NW = NSC * NTILE
M_ALL = N_NODES + B * C
M_A, M_B = 5168, 5152
TOKCH_A = (323, 16)
TOKCH_B = (92, 56)
ENTROWS_H = 26880
ENT_NCH, ENT_CH = 21, 40
NBROWS = B * C * NE
NBROWS_P = 16384
NBE_NCH, NBE_CH = 8, 64
E_NCH, E_CH = 125, 80
ACC_ROWS = 10240
NODES_PER_TILE = ACC_ROWS // NTILE
PREP_BLK = 400


def _prep_body(g_ref, e_ref, wqkv_ref, went_ref, qkv_ref, entp_ref):
    went = went_ref[0:ED] + went_ref[ED:2 * ED] + went_ref[2 * ED:3 * ED]
    qkv_ref[...] = jnp.dot(g_ref[...], wqkv_ref[...],
                           preferred_element_type=jnp.float32)
    entp_ref[...] = jnp.dot(e_ref[...], went,
                            preferred_element_type=jnp.float32)


def _prep_pallas(glove, entity_table, Wqkv, W_ent):
    return pl.pallas_call(
        _prep_body,
        grid=(GV // PREP_BLK,),
        in_specs=[
            pl.BlockSpec((PREP_BLK, WD), lambda i: (i, 0)),
            pl.BlockSpec((PREP_BLK, ED), lambda i: (i, 0)),
            pl.BlockSpec((WD, QKVW), lambda i: (0, 0)),
            pl.BlockSpec((3 * ED, D), lambda i: (0, 0)),
        ],
        out_specs=[
            pl.BlockSpec((PREP_BLK, QKVW), lambda i: (i, 0)),
            pl.BlockSpec((PREP_BLK, D), lambda i: (i, 0)),
        ],
        out_shape=[
            jax.ShapeDtypeStruct((GV, QKVW), jnp.float32),
            jax.ShapeDtypeStruct((EV, D), jnp.float32),
        ],
    )(glove, entity_table, Wqkv, W_ent)


def _make_sc_gather(with_nb, m_half, tok_nch, tok_ch):
    mesh = plsc.VectorSubcoreMesh(core_axis_name="c", subcore_axis_name="s")
    TOK_NCH, TOK_CH = tok_nch, tok_ch
    out_type = [jax.ShapeDtypeStruct((m_half * TPAD, QKVW), jnp.float32),
                jax.ShapeDtypeStruct((ENTROWS_H, D), jnp.float32)]
    scratch = [pltpu.VMEM((TOK_NCH, TOK_CH), jnp.int32),
               pltpu.VMEM((ENT_NCH, ENT_CH), jnp.int32),
               pltpu.VMEM((2, TOK_CH, QKVW), jnp.float32),
               pltpu.VMEM((ENT_CH, D), jnp.float32),
               pltpu.SemaphoreType.DMA,
               pltpu.SemaphoreType.DMA,
               pltpu.SemaphoreType.DMA,
               pltpu.SemaphoreType.DMA]
    if with_nb:
        out_type.append(jax.ShapeDtypeStruct((NBROWS_P, EPAD), jnp.float32))
        scratch = ([pltpu.VMEM((NBE_NCH, NBE_CH), jnp.int32)] + scratch
                   + [pltpu.VMEM((NBE_CH, EPAD), jnp.float32)])

    def body(gq_hbm, ep_hbm, er_hbm, tokidx_hbm, entidx_hbm, nbidx_hbm,
             qkvout_hbm, entout_hbm, nbout_hbm,
             nbidx_v, tokidx_v, entidx_v, tokbuf2, entbuf,
             sem_0, sem_1, sem_w0, sem_w1, nbbuf=None):
        cid = lax.axis_index("c"); sid = lax.axis_index("s")
        wid = sid * NSC + cid
        pltpu.sync_copy(tokidx_hbm.at[wid], tokidx_v)
        pltpu.sync_copy(entidx_hbm.at[wid], entidx_v)
        if nbidx_hbm is not None:
            pltpu.sync_copy(nbidx_hbm.at[wid], nbidx_v)
        tbase = wid * (TOK_NCH * TOK_CH)
        even = TOK_NCH - (TOK_NCH % 2)

        @pl.loop(0, even, step=2)
        def _tok(j):
            ga = pltpu.async_copy(gq_hbm.at[tokidx_v.at[j]], tokbuf2.at[0], sem_0)
            gb = pltpu.async_copy(gq_hbm.at[tokidx_v.at[j + 1]], tokbuf2.at[1], sem_1)
            ga.wait()
            wa = pltpu.async_copy(
                tokbuf2.at[0], qkvout_hbm.at[pl.ds(tbase + j * TOK_CH, TOK_CH)], sem_w0)
            gb.wait()
            wb = pltpu.async_copy(
                tokbuf2.at[1],
                qkvout_hbm.at[pl.ds(tbase + (j + 1) * TOK_CH, TOK_CH)], sem_w1)
            wa.wait()
            wb.wait()

        if TOK_NCH % 2:
            @pl.loop(even, TOK_NCH)
            def _tok_tail(j):
                pltpu.async_copy(gq_hbm.at[tokidx_v.at[j]], tokbuf2.at[0], sem_0).wait()
                pltpu.sync_copy(tokbuf2.at[0],
                                qkvout_hbm.at[pl.ds(tbase + j * TOK_CH, TOK_CH)])

        ebase = wid * (ENT_NCH * ENT_CH)

        @pl.loop(0, ENT_NCH)
        def _ent(j):
            pltpu.async_copy(ep_hbm.at[entidx_v.at[j]], entbuf, sem_0).wait()
            pltpu.sync_copy(entbuf, entout_hbm.at[pl.ds(ebase + j * ENT_CH, ENT_CH)])

        if nbout_hbm is not None:
            nbase = wid * (NBE_NCH * NBE_CH)

            @pl.loop(0, NBE_NCH)
            def _nb(j):
                pltpu.async_copy(er_hbm.at[nbidx_v.at[j]], nbbuf, sem_0).wait()
                pltpu.sync_copy(nbbuf, nbout_hbm.at[pl.ds(nbase + j * NBE_CH, NBE_CH)])

    if with_nb:
        def k_nb(gq, ep, er, tokidx, entidx, nbidx, qkvout, entout, nbout,
                 nbidx_v, tokidx_v, entidx_v, tokbuf2, entbuf,
                 sem_0, sem_1, sem_w0, sem_w1, nbbuf):
            body(gq, ep, er, tokidx, entidx, nbidx, qkvout, entout, nbout,
                 nbidx_v, tokidx_v, entidx_v, tokbuf2, entbuf,
                 sem_0, sem_1, sem_w0, sem_w1, nbbuf)
        return pl.kernel(k_nb, out_type=out_type, mesh=mesh, scratch_types=scratch)

    def k_plain(gq, ep, tokidx, entidx, qkvout, entout,
                tokidx_v, entidx_v, tokbuf2, entbuf, sem_0, sem_1, sem_w0, sem_w1):
        body(gq, ep, None, tokidx, entidx, None, qkvout, entout, None,
             None, tokidx_v, entidx_v, tokbuf2, entbuf, sem_0, sem_1, sem_w0, sem_w1)
    return pl.kernel(k_plain, out_type=out_type, mesh=mesh, scratch_types=scratch)


_sc_gather_half_a = _make_sc_gather(False, M_A, *TOKCH_A)
_sc_gather_half_b = _make_sc_gather(True, M_B, *TOKCH_B)


def _sc_segsum(m, srcidx3, dstidx3, zrows):
    mesh = plsc.VectorSubcoreMesh(core_axis_name="c", subcore_axis_name="s")

    @functools.partial(
        pl.kernel,
        out_type=jax.ShapeDtypeStruct((NSC, ACC_ROWS, D), jnp.float32),
        mesh=mesh,
        scratch_types=[pltpu.VMEM((E_NCH, E_CH), jnp.int32),
                       pltpu.VMEM((E_NCH, E_CH), jnp.int32),
                       pltpu.VMEM((E_CH, D), jnp.float32),
                       pltpu.VMEM_SHARED((ACC_ROWS, D), jnp.float32),
                       pltpu.SemaphoreType.DMA],
    )
    def k(m_hbm, srcidx_hbm, dstidx_hbm, z_hbm, out_hbm,
          sidx_v, didx_v, rows_v, acc, sem):
        cid = lax.axis_index("c"); sid = lax.axis_index("s")
        wid2 = cid * NTILE + sid
        pltpu.sync_copy(srcidx_hbm.at[wid2], sidx_v)
        pltpu.sync_copy(dstidx_hbm.at[wid2], didx_v)
        pltpu.sync_copy(z_hbm, acc.at[pl.ds(sid * NODES_PER_TILE, NODES_PER_TILE)])
        plsc.subcore_barrier()

        @pl.loop(0, E_NCH)
        def _e(j):
            pltpu.async_copy(m_hbm.at[sidx_v.at[j]], rows_v, sem).wait()
            pltpu.sync_copy(rows_v, acc.at[didx_v.at[j]], add=True)

        plsc.subcore_barrier()
        pltpu.sync_copy(
            acc.at[pl.ds(sid * NODES_PER_TILE, NODES_PER_TILE)],
            out_hbm.at[cid].at[pl.ds(sid * NODES_PER_TILE, NODES_PER_TILE)])

    return k(m, srcidx3, dstidx3, zrows)


CLK = NC * B
CLK_PT = 128
CLK_TILES = CLK // CLK_PT


def _sc_gather_clicked(xenc, gemb, map_idx3):
    mesh = plsc.VectorSubcoreMesh(core_axis_name="c", subcore_axis_name="s")

    @functools.partial(
        pl.kernel,
        out_type=[jax.ShapeDtypeStruct((CLK, D), jnp.float32),
                  jax.ShapeDtypeStruct((CLK, D), jnp.float32)],
        mesh=mesh,
        scratch_types=[pltpu.VMEM((1, CLK_PT), jnp.int32),
                       pltpu.VMEM((CLK_PT, D), jnp.float32),
                       pltpu.SemaphoreType.DMA],
    )
    def k(xenc_hbm, gemb_hbm, mapidx_hbm, out1_hbm, out2_hbm, idx_v, buf, sem):
        cid = lax.axis_index("c"); sid = lax.axis_index("s")
        wid = sid * NSC + cid

        @pl.when(wid < CLK_TILES)
        def _():
            pltpu.sync_copy(mapidx_hbm.at[wid], idx_v)
            pltpu.async_copy(xenc_hbm.at[idx_v.at[0]], buf, sem).wait()
            pltpu.sync_copy(buf, out1_hbm.at[pl.ds(wid * CLK_PT, CLK_PT)])
            pltpu.async_copy(gemb_hbm.at[idx_v.at[0]], buf, sem).wait()
            pltpu.sync_copy(buf, out2_hbm.at[pl.ds(wid * CLK_PT, CLK_PT)])

    return k(xenc, gemb, map_idx3)


def _encode_body(qkv_ref, entm_ref, watt_ref, batt_ref, vatt_ref,
                 wggc0_ref, xenc_ref, m0_ref, es_scr, out_scr):
    bf16 = jnp.bfloat16
    qkv = qkv_ref[...]
    rowmod = lax.broadcasted_iota(jnp.int32, (NB * TPAD, QKVW), 0) % TPAD
    qkv = jnp.where(rowmod < NUM_TOK, qkv, 0.0)

    i0 = lax.broadcasted_iota(jnp.int32, (4 * TPAD, D), 0)
    i1 = lax.broadcasted_iota(jnp.int32, (4 * TPAD, D), 1)
    hm0 = (i0 // TPAD == i1 // HD).astype(bf16)
    hm1 = (i0 // TPAD == i1 // HD - 4).astype(bf16)
    o0 = lax.broadcasted_iota(jnp.int32, (H * TPAD, H), 0)
    o1 = lax.broadcasted_iota(jnp.int32, (H * TPAD, H), 1)
    onesbd = ((o0 // TPAD == o1) & (o0 % TPAD < NUM_TOK)).astype(jnp.float32)
    x0 = lax.broadcasted_iota(jnp.int32, (H, D), 0)
    x1 = lax.broadcasted_iota(jnp.int32, (H, D), 1)
    expand = (x0 == x1 // HD).astype(jnp.float32)
    tmask = lax.broadcasted_iota(jnp.int32, (TPAD, 1), 0) < NUM_TOK

    nt = (((1,), (1,)), ((), ()))
    nn = (((1,), (0,)), ((), ()))
    qkvb = qkv.astype(bf16)
    for n in range(NB):
        q = qkvb[n * TPAD:(n + 1) * TPAD, 0:D]
        k = qkvb[n * TPAD:(n + 1) * TPAD, D:2 * D]
        k4 = jnp.concatenate([k] * 4, axis=0)
        s0 = lax.dot_general(q, k4 * hm0, nt, preferred_element_type=jnp.float32)
        s1 = lax.dot_general(q, k4 * hm1, nt, preferred_element_type=jnp.float32)
        es_scr[pl.ds(n * TPAD, TPAD), :] = jnp.concatenate([s0, s1], axis=1)
    sc_all = es_scr[...] * 0.25
    mrow = jnp.max(sc_all, axis=1, keepdims=True)
    es_all = jnp.exp(sc_all - mrow)
    es_scr[...] = es_all
    esb_all = es_all.astype(bf16)
    for n in range(NB):
        v = qkvb[n * TPAD:(n + 1) * TPAD, 2 * D:3 * D]
        v4 = jnp.concatenate([v] * 4, axis=0)
        esb = esb_all[n * TPAD:(n + 1) * TPAD, :]
        ou = (lax.dot_general(esb[:, 0:D], v4 * hm0, nn,
                              preferred_element_type=jnp.float32)
              + lax.dot_general(esb[:, D:2 * D], v4 * hm1, nn,
                                preferred_element_type=jnp.float32))
        out_scr[pl.ds(n * TPAD, TPAD), :] = ou

    denom = lax.dot_general(es_scr[...], onesbd, nn,
                            preferred_element_type=jnp.float32)
    recipb = lax.dot_general(1.0 / denom, expand, nn,
                             preferred_element_type=jnp.float32)
    out_all = out_scr[...] * recipb
    an = jnp.tanh(jnp.dot(out_all, watt_ref[...],
                          preferred_element_type=jnp.float32) + batt_ref[...])
    alv = jnp.sum(an * vatt_ref[...], axis=1, keepdims=True)
    pooled_rows = []
    for n in range(NB):
        al = jnp.where(tmask, alv[n * TPAD:(n + 1) * TPAD, :], -1e30)
        wm = jnp.exp(al - jnp.max(al, axis=0, keepdims=True))
        wm = wm / jnp.sum(wm, axis=0, keepdims=True)
        outn = out_all[n * TPAD:(n + 1) * TPAD, :]
        pooled_rows.append(jnp.sum(outn * wm, axis=0, keepdims=True))
    pooledb = jnp.concatenate(pooled_rows, axis=0)
    entm = jnp.mean(entm_ref[...], axis=1)
    xe = pooledb + entm
    xenc_ref[...] = xe
    m0_ref[...] = jnp.dot(xe, wggc0_ref[...], preferred_element_type=jnp.float32)


def _encode_pallas(qkv_flat, entm, W_att, b_att, v_att, W_ggc0):
    M = entm.shape[0]
    grid = M // NB
    return pl.pallas_call(
        _encode_body,
        grid=(grid,),
        in_specs=[
            pl.BlockSpec((NB * TPAD, QKVW), lambda i: (i, 0)),
            pl.BlockSpec((NB, ES, D), lambda i: (i, 0, 0)),
            pl.BlockSpec((D, D), lambda i: (0, 0)),
            pl.BlockSpec((1, D), lambda i: (0, 0)),
            pl.BlockSpec((1, D), lambda i: (0, 0)),
            pl.BlockSpec((D, D), lambda i: (0, 0)),
        ],
        out_specs=[
            pl.BlockSpec((NB, D), lambda i: (i, 0)),
            pl.BlockSpec((NB, D), lambda i: (i, 0)),
        ],
        out_shape=[
            jax.ShapeDtypeStruct((M, D), jnp.float32),
            jax.ShapeDtypeStruct((M, D), jnp.float32),
        ],
        scratch_shapes=[
            pltpu.VMEM((NB * TPAD, H * TPAD), jnp.float32),
            pltpu.VMEM((NB * TPAD, D), jnp.float32),
        ],
    )(qkv_flat, entm, W_att, b_att, v_att, W_ggc0)


GRU_BLK = 400


def _gru_body(h_ref, agg0_ref, agg1_ref, wih_ref, whh_ref, bih_ref, bhh_ref, wnext_ref,
              hout_ref, mnext_ref):
    h = h_ref[...]
    agg = agg0_ref[0] + agg1_ref[0]
    gi = jnp.dot(agg, wih_ref[...], preferred_element_type=jnp.float32) + bih_ref[...]
    gh = jnp.dot(h, whh_ref[...], preferred_element_type=jnp.float32) + bhh_ref[...]
    r = jax.nn.sigmoid(gi[:, :D] + gh[:, :D])
    z = jax.nn.sigmoid(gi[:, D:2 * D] + gh[:, D:2 * D])
    n = jnp.tanh(gi[:, 2 * D:] + r * gh[:, 2 * D:])
    hn = (1.0 - z) * n + z * h
    hout_ref[...] = hn
    mnext_ref[...] = jnp.dot(hn, wnext_ref[...], preferred_element_type=jnp.float32)


def _gru_pallas(h, agg2, W_ih, W_hh, b_ih, b_hh, W_next):
    M = h.shape[0]
    return pl.pallas_call(
        _gru_body,
        grid=(M // GRU_BLK,),
        in_specs=[
            pl.BlockSpec((GRU_BLK, D), lambda i: (i, 0)),
            pl.BlockSpec((1, GRU_BLK, D), lambda i: (0, i, 0)),
            pl.BlockSpec((1, GRU_BLK, D), lambda i: (1, i, 0)),
            pl.BlockSpec((D, 3 * D), lambda i: (0, 0)),
            pl.BlockSpec((D, 3 * D), lambda i: (0, 0)),
            pl.BlockSpec((1, 3 * D), lambda i: (0, 0)),
            pl.BlockSpec((1, 3 * D), lambda i: (0, 0)),
            pl.BlockSpec((D, D), lambda i: (0, 0)),
        ],
        out_specs=[
            pl.BlockSpec((GRU_BLK, D), lambda i: (i, 0)),
            pl.BlockSpec((GRU_BLK, D), lambda i: (i, 0)),
        ],
        out_shape=[
            jax.ShapeDtypeStruct((M, D), jnp.float32),
            jax.ShapeDtypeStruct((M, D), jnp.float32),
        ],
    )(h, agg2, agg2, W_ih, W_hh, b_ih, b_hh, W_next)


def _lstm_body(x_ref, wih_ref, whh_ref, b_ref, hout_ref):
    wih = wih_ref[...]; whh = whh_ref[...]; bb = b_ref[...]

    def step(t, carry):
        hs, cs = carry
        xt = x_ref[pl.ds(t * B, B), :]
        g = (jnp.dot(xt, wih, preferred_element_type=jnp.float32)
             + jnp.dot(hs, whh, preferred_element_type=jnp.float32) + bb)
        i = jax.nn.sigmoid(g[:, :UD])
        f = jax.nn.sigmoid(g[:, UD:2 * UD])
        gg = jnp.tanh(g[:, 2 * UD:3 * UD])
        o = jax.nn.sigmoid(g[:, 3 * UD:])
        cs = f * cs + i * gg
        hs = o * jnp.tanh(cs)
        return (hs, cs)

    z = jnp.zeros((B, UD), jnp.float32)
    hs, _ = lax.fori_loop(0, NC, step, (z, z))
    hout_ref[...] = hs


def _lstm_pallas(clicked_tm, W_lih, W_lhh, b_l):
    return pl.pallas_call(
        _lstm_body,
        out_shape=jax.ShapeDtypeStruct((B, UD), jnp.float32),
    )(clicked_tm, W_lih, W_lhh, b_l)


NBC = 8


def _candnb_body(ne_ref, maskt_ref, wge_ref, bge_ref, vge_ref, wproj_ref, out_ref):
    ne = ne_ref[...]
    t = jnp.tanh(jnp.dot(ne, wge_ref[...], preferred_element_type=jnp.float32)
                 + bge_ref[...])
    a = jnp.sum(t * vge_ref[...], axis=1, keepdims=True)
    rows = []
    for c in range(NBC):
        ac = a[c * NE:(c + 1) * NE, :]
        mc = maskt_ref[0, :, c:c + 1]
        ac = jnp.where(mc > 0, ac, -1e9)
        wme = jnp.exp(ac - jnp.max(ac, axis=0, keepdims=True))
        wme = wme / jnp.sum(wme, axis=0, keepdims=True)
        nec = ne[c * NE:(c + 1) * NE, :]
        rows.append(jnp.sum(nec * wme, axis=0, keepdims=True))
    wsum = jnp.concatenate(rows, axis=0)
    out_ref[...] = jnp.dot(wsum, wproj_ref[...], preferred_element_type=jnp.float32)


def _candnb_pallas(ne_rows, mask_t, W_ge, b_ge, v_ge, W_gproj):
    M = B * C
    return pl.pallas_call(
        _candnb_body,
        grid=(M // NBC,),
        in_specs=[
            pl.BlockSpec((NBC * NE, EPAD), lambda i: (i, 0)),
            pl.BlockSpec((1, NE, NBC), lambda i: (i, 0, 0)),
            pl.BlockSpec((EPAD, EPAD), lambda i: (0, 0)),
            pl.BlockSpec((1, EPAD), lambda i: (0, 0)),
            pl.BlockSpec((1, EPAD), lambda i: (0, 0)),
            pl.BlockSpec((EPAD, D), lambda i: (0, 0)),
        ],
        out_specs=pl.BlockSpec((NBC, D), lambda i: (i, 0)),
        out_shape=jax.ShapeDtypeStruct((M, D), jnp.float32),
    )(ne_rows, mask_t, W_ge, b_ge, v_ge, W_gproj)


def _score_loss_body(cand_ref, user_ref, label_ref, loss_ref, score_ref):
    cand = cand_ref[...]
    user = user_ref[...]
    score = jnp.sum(cand * user[:, None, :], axis=-1)
    m = jnp.max(score, axis=-1, keepdims=True)
    lse = m + jnp.log(jnp.sum(jnp.exp(score - m), axis=-1, keepdims=True))
    logp = score - lse
    lbl = label_ref[...]
    onehot = (lax.broadcasted_iota(jnp.int32, (B, C), 1) == lbl[:, None]).astype(jnp.float32)
    loss = -jnp.mean(jnp.sum(logp * onehot, axis=-1))
    loss_ref[...] = jnp.broadcast_to(loss, (1, 1))
    score_ref[...] = score


def _score_loss(cand_final, user_emb, label):
    loss, score = pl.pallas_call(
        _score_loss_body,
        out_shape=(jax.ShapeDtypeStruct((1, 1), jnp.float32),
                   jax.ShapeDtypeStruct((B, C), jnp.float32)),
    )(cand_final, user_emb, label)
    return loss[0, 0], score


def kernel(subgraph_x, edge_index, mapping_idx, candidate_news, candidate_entity, entity_mask, label,
           glove, entity_table, Wq, Wk, Wv, W_att, b_att, v_att, W_ent, W_ggc,
           W_ih, W_hh, b_ih, b_hh, W_lih, W_lhh, b_l, W_ge, b_ge, v_ge, W_gproj):
    f32 = jnp.float32
    Wqkv = jnp.concatenate([Wq, Wk, Wv], axis=1)
    ent_pad = jnp.pad(entity_table, ((0, 0), (0, EPAD - ED)))
    tokens_all = jnp.concatenate(
        [subgraph_x[:, :NUM_TOK],
         candidate_news[..., :NUM_TOK].reshape(B * C, NUM_TOK)], axis=0)
    tok_pad = jnp.pad(tokens_all, ((0, 0), (0, TPAD - NUM_TOK)))
    ent_ids = jnp.concatenate(
        [subgraph_x[:, -8:-3],
         candidate_entity[..., :ES].reshape(B * C, ES)], axis=0)

    glove_qkv, ent_proj = _prep_pallas(glove, entity_table, Wqkv, W_ent)

    def tok_half(lo, m, nch, ch):
        return tok_pad[lo:lo + m].reshape(NW, nch, ch)

    def ent_half(lo, m):
        half = ent_ids[lo:lo + m].reshape(-1)
        return jnp.pad(half, (0, ENTROWS_H - m * ES)).reshape(NW, ENT_NCH, ENT_CH)

    neighbor_e = candidate_entity[..., ES:].reshape(B * C * NE)
    nb_idx3 = jnp.pad(neighbor_e, (0, NBROWS_P - NBROWS)).reshape(NW, NBE_NCH, NBE_CH)
    qkv_a, ent_a = _sc_gather_half_a(glove_qkv, ent_proj,
                                     tok_half(0, M_A, *TOKCH_A), ent_half(0, M_A))
    qkv_b, ent_b, ne_rows_p = _sc_gather_half_b(glove_qkv, ent_proj, ent_pad,
                                                tok_half(M_A, M_B, *TOKCH_B),
                                                ent_half(M_A, M_B), nb_idx3)
    ne_rows = ne_rows_p[:NBROWS]

    bargs = (W_att, b_att.reshape(1, D), v_att.reshape(1, D), W_ggc[0])
    x_a, m0_a = _encode_pallas(qkv_a, ent_a[:M_A * ES].reshape(M_A, ES, D), *bargs)
    x_b, m0_b = _encode_pallas(qkv_b, ent_b[:M_B * ES].reshape(M_B, ES, D), *bargs)
    x_all = jnp.concatenate([x_a, x_b], axis=0)
    m0_all = jnp.concatenate([m0_a, m0_b], axis=0)
    x_encoded = x_all[:N_NODES]
    cand_title = x_all[N_NODES:].reshape(B, C, D)

    srcidx3 = edge_index[0].reshape(NW, E_NCH, E_CH)
    dstidx3 = edge_index[1].reshape(NW, E_NCH, E_CH)
    zrows = jnp.zeros((NODES_PER_TILE, D), f32)
    h = x_encoded
    m = m0_all[:N_NODES]
    for l in range(3):
        agg2 = _sc_segsum(m, srcidx3, dstidx3, zrows)
        W_next = W_ggc[l + 1] if l < 2 else jnp.zeros((D, D), f32)
        h, m = _gru_pallas(h, agg2, W_ih, W_hh, b_ih.reshape(1, 3 * D),
                           b_hh.reshape(1, 3 * D), W_next)
    graph_emb = h

    map_idx3 = jnp.pad(mapping_idx.T.reshape(-1),
                       (0, NW * CLK_PT - CLK)).reshape(NW, 1, CLK_PT)
    clk_x, clk_g = _sc_gather_clicked(x_encoded, graph_emb, map_idx3)
    clicked_tm = jnp.concatenate([clk_x, clk_g], axis=-1)
    user_emb = _lstm_pallas(clicked_tm, W_lih, W_lhh, b_l.reshape(1, 4 * UD))

    mask_t = entity_mask.reshape(B * C // NBC, NBC, NE).transpose(0, 2, 1)
    W_ge_pad = jnp.pad(W_ge, ((0, EPAD - ED), (0, EPAD - ED)))
    b_ge_pad = jnp.pad(b_ge, (0, EPAD - ED)).reshape(1, EPAD)
    v_ge_pad = jnp.pad(v_ge, (0, EPAD - ED)).reshape(1, EPAD)
    W_gproj_pad = jnp.pad(W_gproj, ((0, EPAD - ED), (0, 0)))
    cand_nb = _candnb_pallas(ne_rows, mask_t, W_ge_pad, b_ge_pad, v_ge_pad,
                             W_gproj_pad).reshape(B, C, D)

    cand_final = jnp.concatenate([cand_nb, cand_title], axis=-1)
    loss, score = _score_loss(cand_final, user_emb, label)
    return (loss, score)


# speedup vs baseline: 3.5935x; 1.0076x over previous
import functools

import jax
import jax.numpy as jnp
import numpy as np
from jax import lax
from jax.experimental import pallas as pl
from jax.experimental.pallas import tpu as pltpu
from jax.experimental.pallas import tpu_sc as plsc

N_NODES = 10000; N_EDGES = 320000; TOKEN_DIM = 38; NUM_TOK = 30
B = 64; NC = 50; C = 5; ES = 5; EN = 10
GV = 100000; EV = 100000; WD = 300; ED = 100; D = 128; UD = 256; H = 8; HD = 16
TPAD = 32
EPAD = 128
QKVW = 3 * D
NB = 16
NE = ES * EN

NSC = 2; NTILE = 16; NW = NSC * NTILE
M_ALL = N_NODES + B * C
M_A, M_B = 5168, 5152
TOKCH_A = (323, 16)
TOKCH_B = (92, 56)
ENTROWS_H = 26880
ENT_NCH, ENT_CH = 21, 40
NBROWS = B * C * NE
NBROWS_P = 16384
NBE_NCH, NBE_CH = 8, 64
E_NCH, E_CH = 125, 80
ACC_ROWS = 10240
NODES_PER_TILE = ACC_ROWS // NTILE
PREP_BLK = 400


def _prep_body(g_ref, e_ref, wqkv_ref, went_ref, qkv_ref, entp_ref):
    went = went_ref[0:ED] + went_ref[ED:2 * ED] + went_ref[2 * ED:3 * ED]
    qkv_ref[...] = jnp.dot(g_ref[...], wqkv_ref[...],
                           preferred_element_type=jnp.float32)
    entp_ref[...] = jnp.dot(e_ref[...], went,
                            preferred_element_type=jnp.float32)


def _prep_pallas(glove, entity_table, Wqkv, W_ent):
    return pl.pallas_call(
        _prep_body,
        grid=(GV // PREP_BLK,),
        in_specs=[
            pl.BlockSpec((PREP_BLK, WD), lambda i: (i, 0)),
            pl.BlockSpec((PREP_BLK, ED), lambda i: (i, 0)),
            pl.BlockSpec((WD, QKVW), lambda i: (0, 0)),
            pl.BlockSpec((3 * ED, D), lambda i: (0, 0)),
        ],
        out_specs=[
            pl.BlockSpec((PREP_BLK, QKVW), lambda i: (i, 0)),
            pl.BlockSpec((PREP_BLK, D), lambda i: (i, 0)),
        ],
        out_shape=[
            jax.ShapeDtypeStruct((GV, QKVW), jnp.float32),
            jax.ShapeDtypeStruct((EV, D), jnp.float32),
        ],
    )(glove, entity_table, Wqkv, W_ent)


def _make_sc_gather(with_nb, m_half, tok_nch, tok_ch):
    mesh = plsc.VectorSubcoreMesh(core_axis_name="c", subcore_axis_name="s")
    TOK_NCH, TOK_CH = tok_nch, tok_ch
    out_type = [jax.ShapeDtypeStruct((m_half * TPAD, QKVW), jnp.float32),
                jax.ShapeDtypeStruct((ENTROWS_H, D), jnp.float32)]
    scratch = [pltpu.VMEM((TOK_NCH, TOK_CH), jnp.int32),
               pltpu.VMEM((ENT_NCH, ENT_CH), jnp.int32),
               pltpu.VMEM((2, TOK_CH, QKVW), jnp.float32),
               pltpu.VMEM((ENT_CH, D), jnp.float32),
               pltpu.SemaphoreType.DMA,
               pltpu.SemaphoreType.DMA,
               pltpu.SemaphoreType.DMA,
               pltpu.SemaphoreType.DMA]
    if with_nb:
        out_type.append(jax.ShapeDtypeStruct((NBROWS_P, EPAD), jnp.float32))
        scratch = ([pltpu.VMEM((NBE_NCH, NBE_CH), jnp.int32)] + scratch
                   + [pltpu.VMEM((NBE_CH, EPAD), jnp.float32)])

    def body(gq_hbm, ep_hbm, er_hbm, tokidx_hbm, entidx_hbm, nbidx_hbm,
             qkvout_hbm, entout_hbm, nbout_hbm,
             nbidx_v, tokidx_v, entidx_v, tokbuf2, entbuf,
             sem_0, sem_1, sem_w0, sem_w1, nbbuf=None):
        cid = lax.axis_index("c"); sid = lax.axis_index("s")
        wid = sid * NSC + cid
        pltpu.sync_copy(tokidx_hbm.at[wid], tokidx_v)
        pltpu.sync_copy(entidx_hbm.at[wid], entidx_v)
        if nbidx_hbm is not None:
            pltpu.sync_copy(nbidx_hbm.at[wid], nbidx_v)
        tbase = wid * (TOK_NCH * TOK_CH)
        even = TOK_NCH - (TOK_NCH % 2)

        @pl.loop(0, even, step=2)
        def _tok(j):
            ga = pltpu.async_copy(gq_hbm.at[tokidx_v.at[j]], tokbuf2.at[0], sem_0)
            gb = pltpu.async_copy(gq_hbm.at[tokidx_v.at[j + 1]], tokbuf2.at[1], sem_1)
            ga.wait()
            wa = pltpu.async_copy(
                tokbuf2.at[0], qkvout_hbm.at[pl.ds(tbase + j * TOK_CH, TOK_CH)], sem_w0)
            gb.wait()
            wb = pltpu.async_copy(
                tokbuf2.at[1],
                qkvout_hbm.at[pl.ds(tbase + (j + 1) * TOK_CH, TOK_CH)], sem_w1)
            wa.wait()
            wb.wait()

        if TOK_NCH % 2:
            @pl.loop(even, TOK_NCH)
            def _tok_tail(j):
                pltpu.async_copy(gq_hbm.at[tokidx_v.at[j]], tokbuf2.at[0], sem_0).wait()
                pltpu.sync_copy(tokbuf2.at[0],
                                qkvout_hbm.at[pl.ds(tbase + j * TOK_CH, TOK_CH)])

        ebase = wid * (ENT_NCH * ENT_CH)

        @pl.loop(0, ENT_NCH)
        def _ent(j):
            pltpu.async_copy(ep_hbm.at[entidx_v.at[j]], entbuf, sem_0).wait()
            pltpu.sync_copy(entbuf, entout_hbm.at[pl.ds(ebase + j * ENT_CH, ENT_CH)])

        if nbout_hbm is not None:
            nbase = wid * (NBE_NCH * NBE_CH)

            @pl.loop(0, NBE_NCH)
            def _nb(j):
                pltpu.async_copy(er_hbm.at[nbidx_v.at[j]], nbbuf, sem_0).wait()
                pltpu.sync_copy(nbbuf, nbout_hbm.at[pl.ds(nbase + j * NBE_CH, NBE_CH)])

    if with_nb:
        def k_nb(gq, ep, er, tokidx, entidx, nbidx, qkvout, entout, nbout,
                 nbidx_v, tokidx_v, entidx_v, tokbuf2, entbuf,
                 sem_0, sem_1, sem_w0, sem_w1, nbbuf):
            body(gq, ep, er, tokidx, entidx, nbidx, qkvout, entout, nbout,
                 nbidx_v, tokidx_v, entidx_v, tokbuf2, entbuf,
                 sem_0, sem_1, sem_w0, sem_w1, nbbuf)
        return pl.kernel(k_nb, out_type=out_type, mesh=mesh, scratch_types=scratch)

    def k_plain(gq, ep, tokidx, entidx, qkvout, entout,
                tokidx_v, entidx_v, tokbuf2, entbuf, sem_0, sem_1, sem_w0, sem_w1):
        body(gq, ep, None, tokidx, entidx, None, qkvout, entout, None,
             None, tokidx_v, entidx_v, tokbuf2, entbuf, sem_0, sem_1, sem_w0, sem_w1)
    return pl.kernel(k_plain, out_type=out_type, mesh=mesh, scratch_types=scratch)


_sc_gather_half_a = _make_sc_gather(False, M_A, *TOKCH_A)
_sc_gather_half_b = _make_sc_gather(True, M_B, *TOKCH_B)


def _sc_segsum(m, edgeidx4, zrows):
    mesh = plsc.VectorSubcoreMesh(core_axis_name="c", subcore_axis_name="s")

    @functools.partial(
        pl.kernel,
        out_type=jax.ShapeDtypeStruct((NSC, ACC_ROWS, D), jnp.float32),
        mesh=mesh,
        scratch_types=[pltpu.VMEM((2, E_CH), jnp.int32),
                       pltpu.VMEM((2, E_CH), jnp.int32),
                       pltpu.VMEM((2, E_CH, D), jnp.float32),
                       pltpu.VMEM_SHARED((ACC_ROWS, D), jnp.float32),
                       pltpu.SemaphoreType.DMA,
                       pltpu.SemaphoreType.DMA,
                       pltpu.SemaphoreType.DMA,
                       pltpu.SemaphoreType.DMA],
    )
    def k(m_hbm, eidx_hbm, z_hbm, out_hbm,
          ibuf0, ibuf1, rows2, acc, sem0, sem1, semi0, semi1):
        cid = lax.axis_index("c"); sid = lax.axis_index("s")
        wid2 = cid * NTILE + sid
        pltpu.sync_copy(z_hbm, acc.at[pl.ds(sid * NODES_PER_TILE, NODES_PER_TILE)])
        plsc.subcore_barrier()

        @pl.loop(0, E_NCH - 1, step=2)
        def _e(j):
            i0 = pltpu.async_copy(eidx_hbm.at[wid2].at[j], ibuf0, semi0)
            i1 = pltpu.async_copy(eidx_hbm.at[wid2].at[j + 1], ibuf1, semi1)
            i0.wait()
            g0 = pltpu.async_copy(m_hbm.at[ibuf0.at[0]], rows2.at[0], sem0)
            i1.wait()
            g1 = pltpu.async_copy(m_hbm.at[ibuf1.at[0]], rows2.at[1], sem1)
            g0.wait()
            pltpu.sync_copy(rows2.at[0], acc.at[ibuf0.at[1]], add=True)
            g1.wait()
            pltpu.sync_copy(rows2.at[1], acc.at[ibuf1.at[1]], add=True)

        @pl.loop(E_NCH - 1, E_NCH)
        def _etail(j):
            pltpu.async_copy(eidx_hbm.at[wid2].at[j], ibuf0, semi0).wait()
            pltpu.async_copy(m_hbm.at[ibuf0.at[0]], rows2.at[0], sem0).wait()
            pltpu.sync_copy(rows2.at[0], acc.at[ibuf0.at[1]], add=True)

        plsc.subcore_barrier()
        pltpu.sync_copy(
            acc.at[pl.ds(sid * NODES_PER_TILE, NODES_PER_TILE)],
            out_hbm.at[cid].at[pl.ds(sid * NODES_PER_TILE, NODES_PER_TILE)])

    return k(m, edgeidx4, zrows)


CLK = NC * B
CLK_PT = 128
CLK_TILES = CLK // CLK_PT


def _sc_gather_clicked(xenc, gemb, map_idx3):
    mesh = plsc.VectorSubcoreMesh(core_axis_name="c", subcore_axis_name="s")

    @functools.partial(
        pl.kernel,
        out_type=[jax.ShapeDtypeStruct((CLK, D), jnp.float32),
                  jax.ShapeDtypeStruct((CLK, D), jnp.float32)],
        mesh=mesh,
        scratch_types=[pltpu.VMEM((1, CLK_PT), jnp.int32),
                       pltpu.VMEM((CLK_PT, D), jnp.float32),
                       pltpu.SemaphoreType.DMA],
    )
    def k(xenc_hbm, gemb_hbm, mapidx_hbm, out1_hbm, out2_hbm, idx_v, buf, sem):
        cid = lax.axis_index("c"); sid = lax.axis_index("s")
        wid = sid * NSC + cid

        @pl.when(wid < CLK_TILES)
        def _():
            pltpu.sync_copy(mapidx_hbm.at[wid], idx_v)
            pltpu.async_copy(xenc_hbm.at[idx_v.at[0]], buf, sem).wait()
            pltpu.sync_copy(buf, out1_hbm.at[pl.ds(wid * CLK_PT, CLK_PT)])
            pltpu.async_copy(gemb_hbm.at[idx_v.at[0]], buf, sem).wait()
            pltpu.sync_copy(buf, out2_hbm.at[pl.ds(wid * CLK_PT, CLK_PT)])

    return k(xenc, gemb, map_idx3)


def _encode_body(qkv_ref, entm_ref, watt_ref, batt_ref, vatt_ref,
                 wggc0_ref, xenc_ref, m0_ref, es_scr, out_scr):
    bf16 = jnp.bfloat16
    qkv = qkv_ref[...]
    rowmod = lax.broadcasted_iota(jnp.int32, (NB * TPAD, QKVW), 0) % TPAD
    qkv = jnp.where(rowmod < NUM_TOK, qkv, 0.0)

    i0 = lax.broadcasted_iota(jnp.int32, (4 * TPAD, D), 0)
    i1 = lax.broadcasted_iota(jnp.int32, (4 * TPAD, D), 1)
    hm0 = (i0 // TPAD == i1 // HD).astype(bf16)
    hm1 = (i0 // TPAD == i1 // HD - 4).astype(bf16)
    o0 = lax.broadcasted_iota(jnp.int32, (H * TPAD, H), 0)
    o1 = lax.broadcasted_iota(jnp.int32, (H * TPAD, H), 1)
    onesbd = ((o0 // TPAD == o1) & (o0 % TPAD < NUM_TOK)).astype(jnp.float32)
    x0 = lax.broadcasted_iota(jnp.int32, (H, D), 0)
    x1 = lax.broadcasted_iota(jnp.int32, (H, D), 1)
    expand = (x0 == x1 // HD).astype(jnp.float32)
    tmask = lax.broadcasted_iota(jnp.int32, (TPAD, 1), 0) < NUM_TOK

    nt = (((1,), (1,)), ((), ()))
    nn = (((1,), (0,)), ((), ()))
    qkvb = qkv.astype(bf16)
    for n in range(NB):
        q = qkvb[n * TPAD:(n + 1) * TPAD, 0:D]
        k = qkvb[n * TPAD:(n + 1) * TPAD, D:2 * D]
        k4 = jnp.concatenate([k] * 4, axis=0)
        s0 = lax.dot_general(q, k4 * hm0, nt, preferred_element_type=jnp.float32)
        s1 = lax.dot_general(q, k4 * hm1, nt, preferred_element_type=jnp.float32)
        es_scr[pl.ds(n * TPAD, TPAD), :] = jnp.concatenate([s0, s1], axis=1)
    sc_all = es_scr[...] * 0.25
    mrow = jnp.max(sc_all, axis=1, keepdims=True)
    es_all = jnp.exp(sc_all - mrow)
    es_scr[...] = es_all
    esb_all = es_all.astype(bf16)
    for n in range(NB):
        v = qkvb[n * TPAD:(n + 1) * TPAD, 2 * D:3 * D]
        v4 = jnp.concatenate([v] * 4, axis=0)
        esb = esb_all[n * TPAD:(n + 1) * TPAD, :]
        ou = (lax.dot_general(esb[:, 0:D], v4 * hm0, nn,
                              preferred_element_type=jnp.float32)
              + lax.dot_general(esb[:, D:2 * D], v4 * hm1, nn,
                                preferred_element_type=jnp.float32))
        out_scr[pl.ds(n * TPAD, TPAD), :] = ou

    denom = lax.dot_general(es_scr[...], onesbd, nn,
                            preferred_element_type=jnp.float32)
    recipb = lax.dot_general(1.0 / denom, expand, nn,
                             preferred_element_type=jnp.float32)
    out_all = out_scr[...] * recipb
    an = jnp.tanh(jnp.dot(out_all, watt_ref[...],
                          preferred_element_type=jnp.float32) + batt_ref[...])
    alv = jnp.sum(an * vatt_ref[...], axis=1, keepdims=True)
    pooled_rows = []
    for n in range(NB):
        al = jnp.where(tmask, alv[n * TPAD:(n + 1) * TPAD, :], -1e30)
        wm = jnp.exp(al - jnp.max(al, axis=0, keepdims=True))
        wm = wm / jnp.sum(wm, axis=0, keepdims=True)
        outn = out_all[n * TPAD:(n + 1) * TPAD, :]
        pooled_rows.append(jnp.sum(outn * wm, axis=0, keepdims=True))
    pooledb = jnp.concatenate(pooled_rows, axis=0)
    entm = jnp.mean(entm_ref[...], axis=1)
    xe = pooledb + entm
    xenc_ref[...] = xe
    m0_ref[...] = jnp.dot(xe, wggc0_ref[...], preferred_element_type=jnp.float32)


def _encode_pallas(qkv_flat, entm, W_att, b_att, v_att, W_ggc0):
    M = entm.shape[0]
    grid = M // NB
    return pl.pallas_call(
        _encode_body,
        grid=(grid,),
        in_specs=[
            pl.BlockSpec((NB * TPAD, QKVW), lambda i: (i, 0)),
            pl.BlockSpec((NB, ES, D), lambda i: (i, 0, 0)),
            pl.BlockSpec((D, D), lambda i: (0, 0)),
            pl.BlockSpec((1, D), lambda i: (0, 0)),
            pl.BlockSpec((1, D), lambda i: (0, 0)),
            pl.BlockSpec((D, D), lambda i: (0, 0)),
        ],
        out_specs=[
            pl.BlockSpec((NB, D), lambda i: (i, 0)),
            pl.BlockSpec((NB, D), lambda i: (i, 0)),
        ],
        out_shape=[
            jax.ShapeDtypeStruct((M, D), jnp.float32),
            jax.ShapeDtypeStruct((M, D), jnp.float32),
        ],
        scratch_shapes=[
            pltpu.VMEM((NB * TPAD, H * TPAD), jnp.float32),
            pltpu.VMEM((NB * TPAD, D), jnp.float32),
        ],
    )(qkv_flat, entm, W_att, b_att, v_att, W_ggc0)


GRU_BLK = 400


def _gru_body(h_ref, agg0_ref, agg1_ref, wih_ref, whh_ref, bih_ref, bhh_ref, wnext_ref,
              hout_ref, mnext_ref):
    h = h_ref[...]
    agg = agg0_ref[0] + agg1_ref[0]
    gi = jnp.dot(agg, wih_ref[...], preferred_element_type=jnp.float32) + bih_ref[...]
    gh = jnp.dot(h, whh_ref[...], preferred_element_type=jnp.float32) + bhh_ref[...]
    r = jax.nn.sigmoid(gi[:, :D] + gh[:, :D])
    z = jax.nn.sigmoid(gi[:, D:2 * D] + gh[:, D:2 * D])
    n = jnp.tanh(gi[:, 2 * D:] + r * gh[:, 2 * D:])
    hn = (1.0 - z) * n + z * h
    hout_ref[...] = hn
    mnext_ref[...] = jnp.dot(hn, wnext_ref[...], preferred_element_type=jnp.float32)


def _gru_pallas(h, agg2, W_ih, W_hh, b_ih, b_hh, W_next):
    M = h.shape[0]
    return pl.pallas_call(
        _gru_body,
        grid=(M // GRU_BLK,),
        in_specs=[
            pl.BlockSpec((GRU_BLK, D), lambda i: (i, 0)),
            pl.BlockSpec((1, GRU_BLK, D), lambda i: (0, i, 0)),
            pl.BlockSpec((1, GRU_BLK, D), lambda i: (1, i, 0)),
            pl.BlockSpec((D, 3 * D), lambda i: (0, 0)),
            pl.BlockSpec((D, 3 * D), lambda i: (0, 0)),
            pl.BlockSpec((1, 3 * D), lambda i: (0, 0)),
            pl.BlockSpec((1, 3 * D), lambda i: (0, 0)),
            pl.BlockSpec((D, D), lambda i: (0, 0)),
        ],
        out_specs=[
            pl.BlockSpec((GRU_BLK, D), lambda i: (i, 0)),
            pl.BlockSpec((GRU_BLK, D), lambda i: (i, 0)),
        ],
        out_shape=[
            jax.ShapeDtypeStruct((M, D), jnp.float32),
            jax.ShapeDtypeStruct((M, D), jnp.float32),
        ],
    )(h, agg2, agg2, W_ih, W_hh, b_ih, b_hh, W_next)


def _lstm_body(x_ref, wih_ref, whh_ref, b_ref, hout_ref):
    wih = wih_ref[...]; whh = whh_ref[...]; bb = b_ref[...]

    def step(t, carry):
        hs, cs = carry
        xt = x_ref[pl.ds(t * B, B), :]
        g = (jnp.dot(xt, wih, preferred_element_type=jnp.float32)
             + jnp.dot(hs, whh, preferred_element_type=jnp.float32) + bb)
        i = jax.nn.sigmoid(g[:, :UD])
        f = jax.nn.sigmoid(g[:, UD:2 * UD])
        gg = jnp.tanh(g[:, 2 * UD:3 * UD])
        o = jax.nn.sigmoid(g[:, 3 * UD:])
        cs = f * cs + i * gg
        hs = o * jnp.tanh(cs)
        return (hs, cs)

    z = jnp.zeros((B, UD), jnp.float32)
    hs, _ = lax.fori_loop(0, NC, step, (z, z))
    hout_ref[...] = hs


def _lstm_pallas(clicked_tm, W_lih, W_lhh, b_l):
    return pl.pallas_call(
        _lstm_body,
        out_shape=jax.ShapeDtypeStruct((B, UD), jnp.float32),
    )(clicked_tm, W_lih, W_lhh, b_l)


NBC = 8


def _candnb_body(ne_ref, maskt_ref, wge_ref, bge_ref, vge_ref, wproj_ref, out_ref):
    ne = ne_ref[...]
    t = jnp.tanh(jnp.dot(ne, wge_ref[...], preferred_element_type=jnp.float32)
                 + bge_ref[...])
    a = jnp.sum(t * vge_ref[...], axis=1, keepdims=True)
    rows = []
    for c in range(NBC):
        ac = a[c * NE:(c + 1) * NE, :]
        mc = maskt_ref[0, :, c:c + 1]
        ac = jnp.where(mc > 0, ac, -1e9)
        wme = jnp.exp(ac - jnp.max(ac, axis=0, keepdims=True))
        wme = wme / jnp.sum(wme, axis=0, keepdims=True)
        nec = ne[c * NE:(c + 1) * NE, :]
        rows.append(jnp.sum(nec * wme, axis=0, keepdims=True))
    wsum = jnp.concatenate(rows, axis=0)
    out_ref[...] = jnp.dot(wsum, wproj_ref[...], preferred_element_type=jnp.float32)


def _candnb_pallas(ne_rows, mask_t, W_ge, b_ge, v_ge, W_gproj):
    M = B * C
    return pl.pallas_call(
        _candnb_body,
        grid=(M // NBC,),
        in_specs=[
            pl.BlockSpec((NBC * NE, EPAD), lambda i: (i, 0)),
            pl.BlockSpec((1, NE, NBC), lambda i: (i, 0, 0)),
            pl.BlockSpec((EPAD, EPAD), lambda i: (0, 0)),
            pl.BlockSpec((1, EPAD), lambda i: (0, 0)),
            pl.BlockSpec((1, EPAD), lambda i: (0, 0)),
            pl.BlockSpec((EPAD, D), lambda i: (0, 0)),
        ],
        out_specs=pl.BlockSpec((NBC, D), lambda i: (i, 0)),
        out_shape=jax.ShapeDtypeStruct((M, D), jnp.float32),
    )(ne_rows, mask_t, W_ge, b_ge, v_ge, W_gproj)


def _score_loss_body(cand_ref, user_ref, label_ref, loss_ref, score_ref):
    cand = cand_ref[...]
    user = user_ref[...]
    score = jnp.sum(cand * user[:, None, :], axis=-1)
    m = jnp.max(score, axis=-1, keepdims=True)
    lse = m + jnp.log(jnp.sum(jnp.exp(score - m), axis=-1, keepdims=True))
    logp = score - lse
    lbl = label_ref[...]
    onehot = (lax.broadcasted_iota(jnp.int32, (B, C), 1) == lbl[:, None]).astype(jnp.float32)
    loss = -jnp.mean(jnp.sum(logp * onehot, axis=-1))
    loss_ref[...] = jnp.broadcast_to(loss, (1, 1))
    score_ref[...] = score


def _score_loss(cand_final, user_emb, label):
    loss, score = pl.pallas_call(
        _score_loss_body,
        out_shape=(jax.ShapeDtypeStruct((1, 1), jnp.float32),
                   jax.ShapeDtypeStruct((B, C), jnp.float32)),
    )(cand_final, user_emb, label)
    return loss[0, 0], score


def kernel(subgraph_x, edge_index, mapping_idx, candidate_news, candidate_entity, entity_mask, label,
           glove, entity_table, Wq, Wk, Wv, W_att, b_att, v_att, W_ent, W_ggc,
           W_ih, W_hh, b_ih, b_hh, W_lih, W_lhh, b_l, W_ge, b_ge, v_ge, W_gproj):
    f32 = jnp.float32
    Wqkv = jnp.concatenate([Wq, Wk, Wv], axis=1)
    ent_pad = jnp.pad(entity_table, ((0, 0), (0, EPAD - ED)))
    tokens_all = jnp.concatenate(
        [subgraph_x[:, :NUM_TOK],
         candidate_news[..., :NUM_TOK].reshape(B * C, NUM_TOK)], axis=0)
    tok_pad = jnp.pad(tokens_all, ((0, 0), (0, TPAD - NUM_TOK)))
    ent_ids = jnp.concatenate(
        [subgraph_x[:, -8:-3],
         candidate_entity[..., :ES].reshape(B * C, ES)], axis=0)

    glove_qkv, ent_proj = _prep_pallas(glove, entity_table, Wqkv, W_ent)

    def tok_half(lo, m, nch, ch):
        return tok_pad[lo:lo + m].reshape(NW, nch, ch)

    def ent_half(lo, m):
        half = ent_ids[lo:lo + m].reshape(-1)
        return jnp.pad(half, (0, ENTROWS_H - m * ES)).reshape(NW, ENT_NCH, ENT_CH)

    neighbor_e = candidate_entity[..., ES:].reshape(B * C * NE)
    nb_idx3 = jnp.pad(neighbor_e, (0, NBROWS_P - NBROWS)).reshape(NW, NBE_NCH, NBE_CH)
    qkv_a, ent_a = _sc_gather_half_a(glove_qkv, ent_proj,
                                     tok_half(0, M_A, *TOKCH_A), ent_half(0, M_A))
    qkv_b, ent_b, ne_rows_p = _sc_gather_half_b(glove_qkv, ent_proj, ent_pad,
                                                tok_half(M_A, M_B, *TOKCH_B),
                                                ent_half(M_A, M_B), nb_idx3)
    ne_rows = ne_rows_p[:NBROWS]

    bargs = (W_att, b_att.reshape(1, D), v_att.reshape(1, D), W_ggc[0])
    x_a, m0_a = _encode_pallas(qkv_a, ent_a[:M_A * ES].reshape(M_A, ES, D), *bargs)
    x_b, m0_b = _encode_pallas(qkv_b, ent_b[:M_B * ES].reshape(M_B, ES, D), *bargs)
    x_all = jnp.concatenate([x_a, x_b], axis=0)
    m0_all = jnp.concatenate([m0_a, m0_b], axis=0)
    x_encoded = x_all[:N_NODES]
    cand_title = x_all[N_NODES:].reshape(B, C, D)

    edgeidx4 = jnp.concatenate(
        [edge_index[0].reshape(NW, E_NCH, 1, E_CH),
         edge_index[1].reshape(NW, E_NCH, 1, E_CH)], axis=2)
    zrows = jnp.zeros((NODES_PER_TILE, D), f32)
    h = x_encoded
    m = m0_all[:N_NODES]
    for l in range(3):
        agg2 = _sc_segsum(m, edgeidx4, zrows)
        W_next = W_ggc[l + 1] if l < 2 else jnp.zeros((D, D), f32)
        h, m = _gru_pallas(h, agg2, W_ih, W_hh, b_ih.reshape(1, 3 * D),
                           b_hh.reshape(1, 3 * D), W_next)
    graph_emb = h

    map_idx3 = jnp.pad(mapping_idx.T.reshape(-1),
                       (0, NW * CLK_PT - CLK)).reshape(NW, 1, CLK_PT)
    clk_x, clk_g = _sc_gather_clicked(x_encoded, graph_emb, map_idx3)
    clicked_tm = jnp.concatenate([clk_x, clk_g], axis=-1)
    user_emb = _lstm_pallas(clicked_tm, W_lih, W_lhh, b_l.reshape(1, 4 * UD))

    mask_t = entity_mask.reshape(B * C // NBC, NBC, NE).transpose(0, 2, 1)
    W_ge_pad = jnp.pad(W_ge, ((0, EPAD - ED), (0, EPAD - ED)))
    b_ge_pad = jnp.pad(b_ge, (0, EPAD - ED)).reshape(1, EPAD)
    v_ge_pad = jnp.pad(v_ge, (0, EPAD - ED)).reshape(1, EPAD)
    W_gproj_pad = jnp.pad(W_gproj, ((0, EPAD - ED), (0, 0)))
    cand_nb = _candnb_pallas(ne_rows, mask_t, W_ge_pad, b_ge_pad, v_ge_pad,
                             W_gproj_pad).reshape(B, C, D)

    cand_final = jnp.concatenate([cand_nb, cand_title], axis=-1)
    loss, score = _score_loss(cand_final, user_emb, label)
    return (loss, score)


# speedup vs baseline: 3.5968x; 1.0009x over previous
import functools

import jax
import jax.numpy as jnp
import numpy as np
from jax import lax
from jax.experimental import pallas as pl
from jax.experimental.pallas import tpu as pltpu
from jax.experimental.pallas import tpu_sc as plsc

N_NODES = 10000; N_EDGES = 320000; TOKEN_DIM = 38; NUM_TOK = 30
B = 64; NC = 50; C = 5; ES = 5; EN = 10
GV = 100000; EV = 100000; WD = 300; ED = 100; D = 128; UD = 256; H = 8; HD = 16
TPAD = 32
EPAD = 128
QKVW = 3 * D
NB = 16
NE = ES * EN

NSC = 2; NTILE = 16; NW = NSC * NTILE
M_ALL = N_NODES + B * C
M_A, M_B = 5168, 5152
TOKCH_A = (323, 16)
TOKCH_B = (92, 56)
ENTROWS_H = 26880
ENT_NCH, ENT_CH = 21, 40
NBROWS = B * C * NE
NBROWS_P = 16384
NBE_NCH, NBE_CH = 8, 64
E_NCH, E_CH = 125, 80
ACC_ROWS = 10240
NODES_PER_TILE = ACC_ROWS // NTILE
PREP_BLK = 400


def _prep_body(g_ref, e_ref, wqkv_ref, went_ref, qkv_ref, entp_ref):
    went = went_ref[0:ED] + went_ref[ED:2 * ED] + went_ref[2 * ED:3 * ED]
    qkv_ref[...] = jnp.dot(g_ref[...], wqkv_ref[...],
                           preferred_element_type=jnp.float32)
    entp_ref[...] = jnp.dot(e_ref[...], went,
                            preferred_element_type=jnp.float32)


def _prep_pallas(glove, entity_table, Wqkv, W_ent):
    return pl.pallas_call(
        _prep_body,
        grid=(GV // PREP_BLK,),
        in_specs=[
            pl.BlockSpec((PREP_BLK, WD), lambda i: (i, 0)),
            pl.BlockSpec((PREP_BLK, ED), lambda i: (i, 0)),
            pl.BlockSpec((WD, QKVW), lambda i: (0, 0)),
            pl.BlockSpec((3 * ED, D), lambda i: (0, 0)),
        ],
        out_specs=[
            pl.BlockSpec((PREP_BLK, QKVW), lambda i: (i, 0)),
            pl.BlockSpec((PREP_BLK, D), lambda i: (i, 0)),
        ],
        out_shape=[
            jax.ShapeDtypeStruct((GV, QKVW), jnp.float32),
            jax.ShapeDtypeStruct((EV, D), jnp.float32),
        ],
    )(glove, entity_table, Wqkv, W_ent)


def _make_sc_gather(with_nb, m_half, tok_nch, tok_ch):
    mesh = plsc.VectorSubcoreMesh(core_axis_name="c", subcore_axis_name="s")
    TOK_NCH, TOK_CH = tok_nch, tok_ch
    out_type = [jax.ShapeDtypeStruct((m_half * TPAD, QKVW), jnp.float32),
                jax.ShapeDtypeStruct((ENTROWS_H, D), jnp.float32)]
    scratch = [pltpu.VMEM((TOK_NCH, TOK_CH), jnp.int32),
               pltpu.VMEM((ENT_NCH, ENT_CH), jnp.int32),
               pltpu.VMEM((2, TOK_CH, QKVW), jnp.float32),
               pltpu.VMEM((ENT_CH, D), jnp.float32),
               pltpu.SemaphoreType.DMA,
               pltpu.SemaphoreType.DMA,
               pltpu.SemaphoreType.DMA,
               pltpu.SemaphoreType.DMA]
    if with_nb:
        out_type.append(jax.ShapeDtypeStruct((NBROWS_P, EPAD), jnp.float32))
        scratch = ([pltpu.VMEM((NBE_NCH, NBE_CH), jnp.int32)] + scratch
                   + [pltpu.VMEM((NBE_CH, EPAD), jnp.float32)])

    def body(gq_hbm, ep_hbm, er_hbm, tokidx_hbm, entidx_hbm, nbidx_hbm,
             qkvout_hbm, entout_hbm, nbout_hbm,
             nbidx_v, tokidx_v, entidx_v, tokbuf2, entbuf,
             sem_0, sem_1, sem_w0, sem_w1, nbbuf=None):
        cid = lax.axis_index("c"); sid = lax.axis_index("s")
        wid = sid * NSC + cid
        pltpu.sync_copy(tokidx_hbm.at[wid], tokidx_v)
        pltpu.sync_copy(entidx_hbm.at[wid], entidx_v)
        if nbidx_hbm is not None:
            pltpu.sync_copy(nbidx_hbm.at[wid], nbidx_v)
        tbase = wid * (TOK_NCH * TOK_CH)
        even = TOK_NCH - (TOK_NCH % 2)

        @pl.loop(0, even, step=2)
        def _tok(j):
            ga = pltpu.async_copy(gq_hbm.at[tokidx_v.at[j]], tokbuf2.at[0], sem_0)
            gb = pltpu.async_copy(gq_hbm.at[tokidx_v.at[j + 1]], tokbuf2.at[1], sem_1)
            ga.wait()
            wa = pltpu.async_copy(
                tokbuf2.at[0], qkvout_hbm.at[pl.ds(tbase + j * TOK_CH, TOK_CH)], sem_w0)
            gb.wait()
            wb = pltpu.async_copy(
                tokbuf2.at[1],
                qkvout_hbm.at[pl.ds(tbase + (j + 1) * TOK_CH, TOK_CH)], sem_w1)
            wa.wait()
            wb.wait()

        if TOK_NCH % 2:
            @pl.loop(even, TOK_NCH)
            def _tok_tail(j):
                pltpu.async_copy(gq_hbm.at[tokidx_v.at[j]], tokbuf2.at[0], sem_0).wait()
                pltpu.sync_copy(tokbuf2.at[0],
                                qkvout_hbm.at[pl.ds(tbase + j * TOK_CH, TOK_CH)])

        ebase = wid * (ENT_NCH * ENT_CH)

        @pl.loop(0, ENT_NCH)
        def _ent(j):
            pltpu.async_copy(ep_hbm.at[entidx_v.at[j]], entbuf, sem_0).wait()
            pltpu.sync_copy(entbuf, entout_hbm.at[pl.ds(ebase + j * ENT_CH, ENT_CH)])

        if nbout_hbm is not None:
            nbase = wid * (NBE_NCH * NBE_CH)

            @pl.loop(0, NBE_NCH)
            def _nb(j):
                pltpu.async_copy(er_hbm.at[nbidx_v.at[j]], nbbuf, sem_0).wait()
                pltpu.sync_copy(nbbuf, nbout_hbm.at[pl.ds(nbase + j * NBE_CH, NBE_CH)])

    if with_nb:
        def k_nb(gq, ep, er, tokidx, entidx, nbidx, qkvout, entout, nbout,
                 nbidx_v, tokidx_v, entidx_v, tokbuf2, entbuf,
                 sem_0, sem_1, sem_w0, sem_w1, nbbuf):
            body(gq, ep, er, tokidx, entidx, nbidx, qkvout, entout, nbout,
                 nbidx_v, tokidx_v, entidx_v, tokbuf2, entbuf,
                 sem_0, sem_1, sem_w0, sem_w1, nbbuf)
        return pl.kernel(k_nb, out_type=out_type, mesh=mesh, scratch_types=scratch)

    def k_plain(gq, ep, tokidx, entidx, qkvout, entout,
                tokidx_v, entidx_v, tokbuf2, entbuf, sem_0, sem_1, sem_w0, sem_w1):
        body(gq, ep, None, tokidx, entidx, None, qkvout, entout, None,
             None, tokidx_v, entidx_v, tokbuf2, entbuf, sem_0, sem_1, sem_w0, sem_w1)
    return pl.kernel(k_plain, out_type=out_type, mesh=mesh, scratch_types=scratch)


_sc_gather_half_a = _make_sc_gather(False, M_A, *TOKCH_A)
_sc_gather_half_b = _make_sc_gather(True, M_B, *TOKCH_B)


def _sc_segsum(m, edgeidx4, zrows):
    mesh = plsc.VectorSubcoreMesh(core_axis_name="c", subcore_axis_name="s")

    @functools.partial(
        pl.kernel,
        out_type=jax.ShapeDtypeStruct((NSC, ACC_ROWS, D), jnp.float32),
        mesh=mesh,
        scratch_types=[pltpu.VMEM((2, E_CH), jnp.int32),
                       pltpu.VMEM((2, E_CH), jnp.int32),
                       pltpu.VMEM((2, E_CH, D), jnp.float32),
                       pltpu.VMEM_SHARED((ACC_ROWS, D), jnp.float32),
                       pltpu.SemaphoreType.DMA,
                       pltpu.SemaphoreType.DMA,
                       pltpu.SemaphoreType.DMA,
                       pltpu.SemaphoreType.DMA],
    )
    def k(m_hbm, eidx_hbm, z_hbm, out_hbm,
          ibuf0, ibuf1, rows2, acc, sem0, sem1, semi0, semi1):
        cid = lax.axis_index("c"); sid = lax.axis_index("s")
        wid2 = cid * NTILE + sid
        pltpu.sync_copy(z_hbm, acc.at[pl.ds(sid * NODES_PER_TILE, NODES_PER_TILE)])
        plsc.subcore_barrier()

        @pl.loop(0, E_NCH - 1, step=2)
        def _e(j):
            i0 = pltpu.async_copy(eidx_hbm.at[wid2].at[j], ibuf0, semi0)
            i1 = pltpu.async_copy(eidx_hbm.at[wid2].at[j + 1], ibuf1, semi1)
            i0.wait()
            g0 = pltpu.async_copy(m_hbm.at[ibuf0.at[0]], rows2.at[0], sem0)
            i1.wait()
            g1 = pltpu.async_copy(m_hbm.at[ibuf1.at[0]], rows2.at[1], sem1)
            g0.wait()
            pltpu.sync_copy(rows2.at[0], acc.at[ibuf0.at[1]], add=True)
            g1.wait()
            pltpu.sync_copy(rows2.at[1], acc.at[ibuf1.at[1]], add=True)

        @pl.loop(E_NCH - 1, E_NCH)
        def _etail(j):
            pltpu.async_copy(eidx_hbm.at[wid2].at[j], ibuf0, semi0).wait()
            pltpu.async_copy(m_hbm.at[ibuf0.at[0]], rows2.at[0], sem0).wait()
            pltpu.sync_copy(rows2.at[0], acc.at[ibuf0.at[1]], add=True)

        plsc.subcore_barrier()
        pltpu.sync_copy(
            acc.at[pl.ds(sid * NODES_PER_TILE, NODES_PER_TILE)],
            out_hbm.at[cid].at[pl.ds(sid * NODES_PER_TILE, NODES_PER_TILE)])

    return k(m, edgeidx4, zrows)


CLK = NC * B
CLK_PT = 128
CLK_TILES = CLK // CLK_PT


def _sc_gather_clicked(xenc, gemb, map_idx3):
    mesh = plsc.VectorSubcoreMesh(core_axis_name="c", subcore_axis_name="s")

    @functools.partial(
        pl.kernel,
        out_type=[jax.ShapeDtypeStruct((CLK, D), jnp.float32),
                  jax.ShapeDtypeStruct((CLK, D), jnp.float32)],
        mesh=mesh,
        scratch_types=[pltpu.VMEM((1, CLK_PT), jnp.int32),
                       pltpu.VMEM((CLK_PT, D), jnp.float32),
                       pltpu.SemaphoreType.DMA],
    )
    def k(xenc_hbm, gemb_hbm, mapidx_hbm, out1_hbm, out2_hbm, idx_v, buf, sem):
        cid = lax.axis_index("c"); sid = lax.axis_index("s")
        wid = sid * NSC + cid

        @pl.when(wid < CLK_TILES)
        def _():
            pltpu.sync_copy(mapidx_hbm.at[wid], idx_v)
            pltpu.async_copy(xenc_hbm.at[idx_v.at[0]], buf, sem).wait()
            pltpu.sync_copy(buf, out1_hbm.at[pl.ds(wid * CLK_PT, CLK_PT)])
            pltpu.async_copy(gemb_hbm.at[idx_v.at[0]], buf, sem).wait()
            pltpu.sync_copy(buf, out2_hbm.at[pl.ds(wid * CLK_PT, CLK_PT)])

    return k(xenc, gemb, map_idx3)


def _encode_body(qkv_ref, entm_ref, watt_ref, batt_ref, vatt_ref,
                 wggc0_ref, xenc_ref, m0_ref, es_scr, out_scr):
    bf16 = jnp.bfloat16
    qkv = qkv_ref[...]
    rowmod = lax.broadcasted_iota(jnp.int32, (NB * TPAD, QKVW), 0) % TPAD
    qkv = jnp.where(rowmod < NUM_TOK, qkv, 0.0)

    i0 = lax.broadcasted_iota(jnp.int32, (4 * TPAD, D), 0)
    i1 = lax.broadcasted_iota(jnp.int32, (4 * TPAD, D), 1)
    hm0 = (i0 // TPAD == i1 // HD).astype(bf16)
    hm1 = (i0 // TPAD == i1 // HD - 4).astype(bf16)
    o0 = lax.broadcasted_iota(jnp.int32, (H * TPAD, H), 0)
    o1 = lax.broadcasted_iota(jnp.int32, (H * TPAD, H), 1)
    onesbd = ((o0 // TPAD == o1) & (o0 % TPAD < NUM_TOK)).astype(jnp.float32)
    x0 = lax.broadcasted_iota(jnp.int32, (H, D), 0)
    x1 = lax.broadcasted_iota(jnp.int32, (H, D), 1)
    expand = (x0 == x1 // HD).astype(jnp.float32)
    tmask = lax.broadcasted_iota(jnp.int32, (TPAD, 1), 0) < NUM_TOK

    nt = (((1,), (1,)), ((), ()))
    nn = (((1,), (0,)), ((), ()))
    qkvb = qkv.astype(bf16)
    for n in range(NB):
        q = qkvb[n * TPAD:(n + 1) * TPAD, 0:D]
        k = qkvb[n * TPAD:(n + 1) * TPAD, D:2 * D]
        k4 = jnp.concatenate([k] * 4, axis=0)
        s0 = lax.dot_general(q, k4 * hm0, nt, preferred_element_type=jnp.float32)
        s1 = lax.dot_general(q, k4 * hm1, nt, preferred_element_type=jnp.float32)
        es_scr[pl.ds(n * TPAD, TPAD), :] = jnp.concatenate([s0, s1], axis=1)
    sc_all = es_scr[...] * 0.25
    mrow = jnp.max(sc_all, axis=1, keepdims=True)
    es_scr[...] = jnp.exp(sc_all - mrow)
    hm0f = hm0.astype(jnp.float32); hm1f = hm1.astype(jnp.float32)
    for n in range(NB):
        v = qkv[n * TPAD:(n + 1) * TPAD, 2 * D:3 * D]
        v4 = jnp.concatenate([v] * 4, axis=0)
        esn = es_scr[pl.ds(n * TPAD, TPAD), :]
        ou = (lax.dot_general(esn[:, 0:D], v4 * hm0f, nn,
                              preferred_element_type=jnp.float32)
              + lax.dot_general(esn[:, D:2 * D], v4 * hm1f, nn,
                                preferred_element_type=jnp.float32))
        out_scr[pl.ds(n * TPAD, TPAD), :] = ou

    denom = lax.dot_general(es_scr[...], onesbd, nn,
                            preferred_element_type=jnp.float32)
    recipb = lax.dot_general(1.0 / denom, expand, nn,
                             preferred_element_type=jnp.float32)
    out_all = out_scr[...] * recipb
    an = jnp.tanh(jnp.dot(out_all, watt_ref[...],
                          preferred_element_type=jnp.float32) + batt_ref[...])
    alv = jnp.sum(an * vatt_ref[...], axis=1, keepdims=True)
    pooled_rows = []
    for n in range(NB):
        al = jnp.where(tmask, alv[n * TPAD:(n + 1) * TPAD, :], -1e30)
        wm = jnp.exp(al - jnp.max(al, axis=0, keepdims=True))
        wm = wm / jnp.sum(wm, axis=0, keepdims=True)
        outn = out_all[n * TPAD:(n + 1) * TPAD, :]
        pooled_rows.append(jnp.sum(outn * wm, axis=0, keepdims=True))
    pooledb = jnp.concatenate(pooled_rows, axis=0)
    entm = jnp.mean(entm_ref[...], axis=1)
    xe = pooledb + entm
    xenc_ref[...] = xe
    m0_ref[...] = jnp.dot(xe, wggc0_ref[...], preferred_element_type=jnp.float32)


def _encode_pallas(qkv_flat, entm, W_att, b_att, v_att, W_ggc0):
    M = entm.shape[0]
    grid = M // NB
    return pl.pallas_call(
        _encode_body,
        grid=(grid,),
        in_specs=[
            pl.BlockSpec((NB * TPAD, QKVW), lambda i: (i, 0)),
            pl.BlockSpec((NB, ES, D), lambda i: (i, 0, 0)),
            pl.BlockSpec((D, D), lambda i: (0, 0)),
            pl.BlockSpec((1, D), lambda i: (0, 0)),
            pl.BlockSpec((1, D), lambda i: (0, 0)),
            pl.BlockSpec((D, D), lambda i: (0, 0)),
        ],
        out_specs=[
            pl.BlockSpec((NB, D), lambda i: (i, 0)),
            pl.BlockSpec((NB, D), lambda i: (i, 0)),
        ],
        out_shape=[
            jax.ShapeDtypeStruct((M, D), jnp.float32),
            jax.ShapeDtypeStruct((M, D), jnp.float32),
        ],
        scratch_shapes=[
            pltpu.VMEM((NB * TPAD, H * TPAD), jnp.float32),
            pltpu.VMEM((NB * TPAD, D), jnp.float32),
        ],
    )(qkv_flat, entm, W_att, b_att, v_att, W_ggc0)


GRU_BLK = 400


def _gru_body(h_ref, agg0_ref, agg1_ref, wih_ref, whh_ref, bih_ref, bhh_ref, wnext_ref,
              hout_ref, mnext_ref):
    h = h_ref[...]
    agg = agg0_ref[0] + agg1_ref[0]
    gi = jnp.dot(agg, wih_ref[...], preferred_element_type=jnp.float32) + bih_ref[...]
    gh = jnp.dot(h, whh_ref[...], preferred_element_type=jnp.float32) + bhh_ref[...]
    r = jax.nn.sigmoid(gi[:, :D] + gh[:, :D])
    z = jax.nn.sigmoid(gi[:, D:2 * D] + gh[:, D:2 * D])
    n = jnp.tanh(gi[:, 2 * D:] + r * gh[:, 2 * D:])
    hn = (1.0 - z) * n + z * h
    hout_ref[...] = hn
    mnext_ref[...] = jnp.dot(hn, wnext_ref[...], preferred_element_type=jnp.float32)


def _gru_pallas(h, agg2, W_ih, W_hh, b_ih, b_hh, W_next):
    M = h.shape[0]
    return pl.pallas_call(
        _gru_body,
        grid=(M // GRU_BLK,),
        in_specs=[
            pl.BlockSpec((GRU_BLK, D), lambda i: (i, 0)),
            pl.BlockSpec((1, GRU_BLK, D), lambda i: (0, i, 0)),
            pl.BlockSpec((1, GRU_BLK, D), lambda i: (1, i, 0)),
            pl.BlockSpec((D, 3 * D), lambda i: (0, 0)),
            pl.BlockSpec((D, 3 * D), lambda i: (0, 0)),
            pl.BlockSpec((1, 3 * D), lambda i: (0, 0)),
            pl.BlockSpec((1, 3 * D), lambda i: (0, 0)),
            pl.BlockSpec((D, D), lambda i: (0, 0)),
        ],
        out_specs=[
            pl.BlockSpec((GRU_BLK, D), lambda i: (i, 0)),
            pl.BlockSpec((GRU_BLK, D), lambda i: (i, 0)),
        ],
        out_shape=[
            jax.ShapeDtypeStruct((M, D), jnp.float32),
            jax.ShapeDtypeStruct((M, D), jnp.float32),
        ],
    )(h, agg2, agg2, W_ih, W_hh, b_ih, b_hh, W_next)


def _lstm_body(x_ref, wih_ref, whh_ref, b_ref, hout_ref):
    wih = wih_ref[...]; whh = whh_ref[...]; bb = b_ref[...]

    def step(t, carry):
        hs, cs = carry
        xt = x_ref[pl.ds(t * B, B), :]
        g = (jnp.dot(xt, wih, preferred_element_type=jnp.float32)
             + jnp.dot(hs, whh, preferred_element_type=jnp.float32) + bb)
        i = jax.nn.sigmoid(g[:, :UD])
        f = jax.nn.sigmoid(g[:, UD:2 * UD])
        gg = jnp.tanh(g[:, 2 * UD:3 * UD])
        o = jax.nn.sigmoid(g[:, 3 * UD:])
        cs = f * cs + i * gg
        hs = o * jnp.tanh(cs)
        return (hs, cs)

    z = jnp.zeros((B, UD), jnp.float32)
    hs, _ = lax.fori_loop(0, NC, step, (z, z))
    hout_ref[...] = hs


def _lstm_pallas(clicked_tm, W_lih, W_lhh, b_l):
    return pl.pallas_call(
        _lstm_body,
        out_shape=jax.ShapeDtypeStruct((B, UD), jnp.float32),
    )(clicked_tm, W_lih, W_lhh, b_l)


NBC = 8


def _candnb_body(ne_ref, maskt_ref, wge_ref, bge_ref, vge_ref, wproj_ref, out_ref):
    ne = ne_ref[...]
    t = jnp.tanh(jnp.dot(ne, wge_ref[...], preferred_element_type=jnp.float32)
                 + bge_ref[...])
    a = jnp.sum(t * vge_ref[...], axis=1, keepdims=True)
    rows = []
    for c in range(NBC):
        ac = a[c * NE:(c + 1) * NE, :]
        mc = maskt_ref[0, :, c:c + 1]
        ac = jnp.where(mc > 0, ac, -1e9)
        wme = jnp.exp(ac - jnp.max(ac, axis=0, keepdims=True))
        wme = wme / jnp.sum(wme, axis=0, keepdims=True)
        nec = ne[c * NE:(c + 1) * NE, :]
        rows.append(jnp.sum(nec * wme, axis=0, keepdims=True))
    wsum = jnp.concatenate(rows, axis=0)
    out_ref[...] = jnp.dot(wsum, wproj_ref[...], preferred_element_type=jnp.float32)


def _candnb_pallas(ne_rows, mask_t, W_ge, b_ge, v_ge, W_gproj):
    M = B * C
    return pl.pallas_call(
        _candnb_body,
        grid=(M // NBC,),
        in_specs=[
            pl.BlockSpec((NBC * NE, EPAD), lambda i: (i, 0)),
            pl.BlockSpec((1, NE, NBC), lambda i: (i, 0, 0)),
            pl.BlockSpec((EPAD, EPAD), lambda i: (0, 0)),
            pl.BlockSpec((1, EPAD), lambda i: (0, 0)),
            pl.BlockSpec((1, EPAD), lambda i: (0, 0)),
            pl.BlockSpec((EPAD, D), lambda i: (0, 0)),
        ],
        out_specs=pl.BlockSpec((NBC, D), lambda i: (i, 0)),
        out_shape=jax.ShapeDtypeStruct((M, D), jnp.float32),
    )(ne_rows, mask_t, W_ge, b_ge, v_ge, W_gproj)


def _score_loss_body(cand_ref, user_ref, label_ref, loss_ref, score_ref):
    cand = cand_ref[...]
    user = user_ref[...]
    score = jnp.sum(cand * user[:, None, :], axis=-1)
    m = jnp.max(score, axis=-1, keepdims=True)
    lse = m + jnp.log(jnp.sum(jnp.exp(score - m), axis=-1, keepdims=True))
    logp = score - lse
    lbl = label_ref[...]
    onehot = (lax.broadcasted_iota(jnp.int32, (B, C), 1) == lbl[:, None]).astype(jnp.float32)
    loss = -jnp.mean(jnp.sum(logp * onehot, axis=-1))
    loss_ref[...] = jnp.broadcast_to(loss, (1, 1))
    score_ref[...] = score


def _score_loss(cand_final, user_emb, label):
    loss, score = pl.pallas_call(
        _score_loss_body,
        out_shape=(jax.ShapeDtypeStruct((1, 1), jnp.float32),
                   jax.ShapeDtypeStruct((B, C), jnp.float32)),
    )(cand_final, user_emb, label)
    return loss[0, 0], score


def kernel(subgraph_x, edge_index, mapping_idx, candidate_news, candidate_entity, entity_mask, label,
           glove, entity_table, Wq, Wk, Wv, W_att, b_att, v_att, W_ent, W_ggc,
           W_ih, W_hh, b_ih, b_hh, W_lih, W_lhh, b_l, W_ge, b_ge, v_ge, W_gproj):
    f32 = jnp.float32
    Wqkv = jnp.concatenate([Wq, Wk, Wv], axis=1)
    ent_pad = jnp.pad(entity_table, ((0, 0), (0, EPAD - ED)))
    tokens_all = jnp.concatenate(
        [subgraph_x[:, :NUM_TOK],
         candidate_news[..., :NUM_TOK].reshape(B * C, NUM_TOK)], axis=0)
    tok_pad = jnp.pad(tokens_all, ((0, 0), (0, TPAD - NUM_TOK)))
    ent_ids = jnp.concatenate(
        [subgraph_x[:, -8:-3],
         candidate_entity[..., :ES].reshape(B * C, ES)], axis=0)

    glove_qkv, ent_proj = _prep_pallas(glove, entity_table, Wqkv, W_ent)

    def tok_half(lo, m, nch, ch):
        return tok_pad[lo:lo + m].reshape(NW, nch, ch)

    def ent_half(lo, m):
        half = ent_ids[lo:lo + m].reshape(-1)
        return jnp.pad(half, (0, ENTROWS_H - m * ES)).reshape(NW, ENT_NCH, ENT_CH)

    neighbor_e = candidate_entity[..., ES:].reshape(B * C * NE)
    nb_idx3 = jnp.pad(neighbor_e, (0, NBROWS_P - NBROWS)).reshape(NW, NBE_NCH, NBE_CH)
    qkv_a, ent_a = _sc_gather_half_a(glove_qkv, ent_proj,
                                     tok_half(0, M_A, *TOKCH_A), ent_half(0, M_A))
    qkv_b, ent_b, ne_rows_p = _sc_gather_half_b(glove_qkv, ent_proj, ent_pad,
                                                tok_half(M_A, M_B, *TOKCH_B),
                                                ent_half(M_A, M_B), nb_idx3)
    ne_rows = ne_rows_p[:NBROWS]

    bargs = (W_att, b_att.reshape(1, D), v_att.reshape(1, D), W_ggc[0])
    x_a, m0_a = _encode_pallas(qkv_a, ent_a[:M_A * ES].reshape(M_A, ES, D), *bargs)
    x_b, m0_b = _encode_pallas(qkv_b, ent_b[:M_B * ES].reshape(M_B, ES, D), *bargs)
    x_all = jnp.concatenate([x_a, x_b], axis=0)
    m0_all = jnp.concatenate([m0_a, m0_b], axis=0)
    x_encoded = x_all[:N_NODES]
    cand_title = x_all[N_NODES:].reshape(B, C, D)

    edgeidx4 = jnp.concatenate(
        [edge_index[0].reshape(NW, E_NCH, 1, E_CH),
         edge_index[1].reshape(NW, E_NCH, 1, E_CH)], axis=2)
    zrows = jnp.zeros((NODES_PER_TILE, D), f32)
    h = x_encoded
    m = m0_all[:N_NODES]
    for l in range(3):
        agg2 = _sc_segsum(m, edgeidx4, zrows)
        W_next = W_ggc[l + 1] if l < 2 else jnp.zeros((D, D), f32)
        h, m = _gru_pallas(h, agg2, W_ih, W_hh, b_ih.reshape(1, 3 * D),
                           b_hh.reshape(1, 3 * D), W_next)
    graph_emb = h

    map_idx3 = jnp.pad(mapping_idx.T.reshape(-1),
                       (0, NW * CLK_PT - CLK)).reshape(NW, 1, CLK_PT)
    clk_x, clk_g = _sc_gather_clicked(x_encoded, graph_emb, map_idx3)
    clicked_tm = jnp.concatenate([clk_x, clk_g], axis=-1)
    user_emb = _lstm_pallas(clicked_tm, W_lih, W_lhh, b_l.reshape(1, 4 * UD))

    mask_t = entity_mask.reshape(B * C // NBC, NBC, NE).transpose(0, 2, 1)
    W_ge_pad = jnp.pad(W_ge, ((0, EPAD - ED), (0, EPAD - ED)))
    b_ge_pad = jnp.pad(b_ge, (0, EPAD - ED)).reshape(1, EPAD)
    v_ge_pad = jnp.pad(v_ge, (0, EPAD - ED)).reshape(1, EPAD)
    W_gproj_pad = jnp.pad(W_gproj, ((0, EPAD - ED), (0, 0)))
    cand_nb = _candnb_pallas(ne_rows, mask_t, W_ge_pad, b_ge_pad, v_ge_pad,
                             W_gproj_pad).reshape(B, C, D)

    cand_final = jnp.concatenate([cand_nb, cand_title], axis=-1)
    loss, score = _score_loss(cand_final, user_emb, label)
    return (loss, score)
